# Initial kernel scaffold; baseline (speedup 1.0000x reference)
import jax
import jax.numpy as jnp
from jax.experimental import pallas as pl


def kernel(entity_emb, user_emb, edge_index, edge_type, interact_rows, interact_cols, interact_values, weight):
    raise NotImplementedError("write your pallas kernel here")


# speedup vs baseline: 4.7683x; 4.7683x over previous
import functools

import jax
import jax.numpy as jnp
from jax import lax
from jax.experimental import pallas as pl
from jax.experimental.pallas import tpu as pltpu
from jax.experimental.pallas import tpu_sc as plsc

NE = 50000
NU = 20000
D = 64
R = 23
NC, NS = 2, 16
NW = NC * NS

E_REAL = 800000
E_PAD = 819200
EDGES_PER_SC_WORKER = E_PAD // NS
EB = 128
N_EBATCH = EDGES_PER_SC_WORKER // EB
HALF = D // 2
ENT_OUT_ROWS = 50176
ACC_ROWS = 50304
QROWS = 50176

NNZ_REAL = 500000
NNZ_PAD = 512000
NNZ_PER_WORKER = NNZ_PAD // NW
UB = 128
N_UBATCH = NNZ_PER_WORKER // UB
U_BLOCKS = 157
U_ROWS = U_BLOCKS * 128


def _q_body(x_ref, w_ref, o_ref):
    x = x_ref[...]
    w = w_ref[...]
    o_ref[...] = lax.dot_general(x * x, w * w, (((1,), (1,)), ((), ())),
                                 preferred_element_type=jnp.float32)


def _compute_q(emb_pad, wpad):
    blk = 1792
    return pl.pallas_call(
        _q_body,
        grid=(QROWS // blk,),
        in_specs=[
            pl.BlockSpec((blk, D), lambda i: (i, 0)),
            pl.BlockSpec((128, D), lambda i: (0, 0)),
        ],
        out_specs=pl.BlockSpec((blk, 128), lambda i: (i, 0)),
        out_shape=jax.ShapeDtypeStruct((QROWS, 128), jnp.float32),
    )(emb_pad, wpad)


def _entity_sc_body(qflat_hbm, emb2_hbm, head_hbm, tail_hbm, y_hbm, w2_hbm,
                    zr_hbm, zd_hbm,
                    num_hbm, den_hbm,
                    acc_sh, den_sh,
                    hbuf, tbuf, ybuf, qih, qit, qhv, qtv, ewb,
                    trows, orows, wtab):
    cid = lax.axis_index("c")
    sid = lax.axis_index("s")

    pltpu.sync_copy(w2_hbm.at[cid], wtab)

    for j in range(25):
        b = j * 16 + sid
        @pl.when(b < ACC_ROWS // 128)
        def _():
            pltpu.sync_copy(zr_hbm, acc_sh.at[pl.ds(b * 128, 128)])
            pltpu.sync_copy(zd_hbm, den_sh.at[pl.ds(b * 128, 128)])
    plsc.subcore_barrier()

    def batch_body(bi, carry):
        off = sid * EDGES_PER_SC_WORKER + bi * EB
        pltpu.sync_copy(head_hbm.at[pl.ds(off, EB)], hbuf)
        pltpu.sync_copy(tail_hbm.at[pl.ds(off, EB)], tbuf)
        pltpu.sync_copy(y_hbm.at[pl.ds(off, EB)], ybuf)
        kregs = []
        for c in range(EB // 16):
            s = pl.ds(c * 16, 16)
            h = hbuf[s]
            t = tbuf[s]
            y = ybuf[s]
            k = jnp.where(y == 0, 22, y - 1)
            kregs.append(k)
            qih[s] = h * 128 + k
            qit[s] = t * 128 + k
        pltpu.sync_copy(qflat_hbm.at[qih], qhv)
        pltpu.sync_copy(qflat_hbm.at[qit], qtv)
        pltpu.sync_copy(emb2_hbm.at[cid].at[tbuf], trows)
        ewregs = []
        for c in range(EB // 16):
            s = pl.ds(c * 16, 16)
            ew = jnp.exp(qhv[s] * qtv[s])
            ewregs.append(ew)
            ewb[s] = ew
        for c in range(EB // 16):
            ewc = ewregs[c]
            kc = kregs[c]
            for l in range(16):
                e = c * 16 + l
                ewv = jnp.full((16,), ewc[l], jnp.float32)
                kbase = kc[l] * HALF
                for j in range(HALF // 16):
                    tj = trows[e, pl.ds(j * 16, 16)]
                    rj = wtab[pl.ds(kbase + j * 16, 16)]
                    orows[e, pl.ds(j * 16, 16)] = ewv * (tj * rj)
        pltpu.sync_copy(orows, acc_sh.at[hbuf], add=True)
        pltpu.sync_copy(ewb, den_sh.at[hbuf], add=True)
        return carry

    lax.fori_loop(0, N_EBATCH, batch_body, 0)
    plsc.subcore_barrier()

    for jb in range(25):
        b = jb * 16 + sid
        @pl.when(b < ENT_OUT_ROWS // 128)
        def _():
            pltpu.sync_copy(acc_sh.at[pl.ds(b * 128, 128)],
                            num_hbm.at[cid].at[pl.ds(b * 128, 128)])
            pltpu.sync_copy(den_sh.at[pl.ds(b * 128, 128)],
                            den_hbm.at[cid].at[pl.ds(b * 128, 128)])


def _entity_sc(qflat, emb2, head_p, tail_p, y_p, w2, zr, zd):
    mesh = plsc.VectorSubcoreMesh(core_axis_name="c", subcore_axis_name="s",
                                  num_cores=NC, num_subcores=NS)
    f = pl.kernel(
        _entity_sc_body,
        out_type=(
            jax.ShapeDtypeStruct((NC, ENT_OUT_ROWS, HALF), jnp.float32),
            jax.ShapeDtypeStruct((NC, ENT_OUT_ROWS), jnp.float32),
        ),
        mesh=mesh,
        compiler_params=pltpu.CompilerParams(use_tc_tiling_on_sc=False),
        scratch_types=[
            pltpu.VMEM_SHARED((ACC_ROWS, HALF), jnp.float32),
            pltpu.VMEM_SHARED((ACC_ROWS,), jnp.float32),
            pltpu.VMEM((EB,), jnp.int32),
            pltpu.VMEM((EB,), jnp.int32),
            pltpu.VMEM((EB,), jnp.int32),
            pltpu.VMEM((EB,), jnp.int32),
            pltpu.VMEM((EB,), jnp.int32),
            pltpu.VMEM((EB,), jnp.float32),
            pltpu.VMEM((EB,), jnp.float32),
            pltpu.VMEM((EB,), jnp.float32),
            pltpu.VMEM((EB, HALF), jnp.float32),
            pltpu.VMEM((EB, HALF), jnp.float32),
            pltpu.VMEM((R * HALF,), jnp.float32),
        ],
    )
    return f(qflat, emb2, head_p, tail_p, y_p, w2, zr, zd)


def _div_body(num_ref, den_ref, o_ref):
    num = num_ref[...]
    den = den_ref[...]
    inv0 = (1.0 / jnp.maximum(den[0], 1e-37))[:, None]
    inv1 = (1.0 / jnp.maximum(den[1], 1e-37))[:, None]
    o_ref[...] = jnp.concatenate([num[0] * inv0, num[1] * inv1], axis=1)


def _entity_div(num, den):
    blk = 1792
    return pl.pallas_call(
        _div_body,
        grid=(ENT_OUT_ROWS // blk,),
        in_specs=[
            pl.BlockSpec((NC, blk, HALF), lambda i: (0, i, 0)),
            pl.BlockSpec((NC, blk), lambda i: (0, i)),
        ],
        out_specs=pl.BlockSpec((blk, D), lambda i: (i, 0)),
        out_shape=jax.ShapeDtypeStruct((ENT_OUT_ROWS, D), jnp.float32),
    )(num, den)


def _user_sc_body(emb_hbm, cols_hbm, rows_hbm, vals_hbm, zu_hbm, out_hbm,
                  uacc_sh, cbuf, rbuf, vbuf, erows, orows):
    cid = lax.axis_index("c")
    sid = lax.axis_index("s")
    wid = sid * NC + cid

    for j in range(10):
        b = j * 16 + sid
        @pl.when(b < U_BLOCKS)
        def _():
            pltpu.sync_copy(zu_hbm, uacc_sh.at[pl.ds(b * 128, 128)])
    plsc.subcore_barrier()

    def batch_body(bi, carry):
        off = wid * NNZ_PER_WORKER + bi * UB
        pltpu.sync_copy(cols_hbm.at[pl.ds(off, UB)], cbuf)
        pltpu.sync_copy(rows_hbm.at[pl.ds(off, UB)], rbuf)
        pltpu.sync_copy(vals_hbm.at[pl.ds(off, UB)], vbuf)
        pltpu.sync_copy(emb_hbm.at[cbuf], erows)
        for c in range(UB // 16):
            vc = vbuf[pl.ds(c * 16, 16)]
            for l in range(16):
                e = c * 16 + l
                vv = jnp.full((16,), vc[l], jnp.float32)
                for j in range(D // 16):
                    orows[e, pl.ds(j * 16, 16)] = vv * erows[e, pl.ds(j * 16, 16)]
        pltpu.sync_copy(orows, uacc_sh.at[rbuf], add=True)
        return carry

    lax.fori_loop(0, N_UBATCH, batch_body, 0)
    plsc.subcore_barrier()

    for j in range(10):
        b = j * 16 + sid
        @pl.when(b < U_BLOCKS)
        def _():
            pltpu.sync_copy(uacc_sh.at[pl.ds(b * 128, 128)],
                            out_hbm.at[cid].at[pl.ds(b * 128, 128)])


def _user_sc(emb, cols_p, rows_p, vals_p, zu):
    mesh = plsc.VectorSubcoreMesh(core_axis_name="c", subcore_axis_name="s",
                                  num_cores=NC, num_subcores=NS)
    f = pl.kernel(
        _user_sc_body,
        out_type=jax.ShapeDtypeStruct((NC, U_ROWS, D), jnp.float32),
        mesh=mesh,
        compiler_params=pltpu.CompilerParams(use_tc_tiling_on_sc=False),
        scratch_types=[
            pltpu.VMEM_SHARED((U_ROWS, D), jnp.float32),
            pltpu.VMEM((UB,), jnp.int32),
            pltpu.VMEM((UB,), jnp.int32),
            pltpu.VMEM((UB,), jnp.float32),
            pltpu.VMEM((UB, D), jnp.float32),
            pltpu.VMEM((UB, D), jnp.float32),
        ],
    )
    return f(emb, cols_p, rows_p, vals_p, zu)


def _ep_body(ue_ref, w_ref, p_ref, o_ref):
    ue = ue_ref[...]
    w = w_ref[...]
    logits = lax.dot_general(ue, w, (((1,), (1,)), ((), ())),
                             preferred_element_type=jnp.float32)
    col = lax.broadcasted_iota(jnp.int32, logits.shape, 1)
    logits = jnp.where(col < R, logits, -1e30)
    m = jnp.max(logits, axis=-1, keepdims=True)
    ex = jnp.exp(logits - m)
    score = ex / jnp.sum(ex, axis=-1, keepdims=True)
    mult = jnp.dot(score, w, preferred_element_type=jnp.float32)
    p = p_ref[0] + p_ref[1]
    o_ref[...] = p * (1.0 + mult)


def _epilogue(user_emb, w24, partials):
    blk = 1000
    return pl.pallas_call(
        _ep_body,
        grid=(NU // blk,),
        in_specs=[
            pl.BlockSpec((blk, D), lambda i: (i, 0)),
            pl.BlockSpec((24, D), lambda i: (0, 0)),
            pl.BlockSpec((NC, blk, D), lambda i: (0, i, 0)),
        ],
        out_specs=pl.BlockSpec((blk, D), lambda i: (i, 0)),
        out_shape=jax.ShapeDtypeStruct((NU, D), jnp.float32),
    )(user_emb, w24, partials)


def kernel(entity_emb, user_emb, edge_index, edge_type, interact_rows,
           interact_cols, interact_values, weight):
    i32 = jnp.int32
    head = edge_index[0].astype(i32)
    tail = edge_index[1].astype(i32)
    etype = edge_type.astype(i32)

    emb_pad = jnp.pad(entity_emb, ((0, QROWS - NE), (0, 0)))
    wpad = jnp.pad(weight, ((0, 128 - R), (0, 0)))
    q = _compute_q(emb_pad, wpad)
    qflat = q.reshape(-1)

    npad = E_PAD - E_REAL
    head_p = jnp.concatenate([head, jnp.full((npad,), NE, i32)])
    tail_p = jnp.concatenate([tail, jnp.zeros((npad,), i32)])
    y_p = jnp.concatenate([etype, jnp.ones((npad,), i32)])

    emb2 = jnp.stack([entity_emb[:, :HALF], entity_emb[:, HALF:]])
    w2 = jnp.stack([weight[:, :HALF].reshape(-1), weight[:, HALF:].reshape(-1)])

    zr = jnp.zeros((128, HALF), jnp.float32)
    zd = jnp.zeros((128,), jnp.float32)
    num, den = _entity_sc(qflat, emb2, head_p, tail_p, y_p, w2, zr, zd)
    entity_agg = _entity_div(num, den)[:NE]

    upad = NNZ_PAD - NNZ_REAL
    cols_p = jnp.concatenate([interact_cols.astype(i32), jnp.zeros((upad,), i32)])
    rows_p = jnp.concatenate([interact_rows.astype(i32), jnp.zeros((upad,), i32)])
    vals_p = jnp.concatenate([interact_values, jnp.zeros((upad,), jnp.float32)])
    zu = jnp.zeros((128, D), jnp.float32)
    partials = _user_sc(entity_emb, cols_p, rows_p, vals_p, zu)

    w24 = jnp.pad(weight, ((0, 1), (0, 0)))
    user_agg = _epilogue(user_emb, w24, partials)
    return (entity_agg, user_agg)


# speedup vs baseline: 8.6797x; 1.8203x over previous
import functools

import jax
import jax.numpy as jnp
from jax import lax
from jax.experimental import pallas as pl
from jax.experimental.pallas import tpu as pltpu
from jax.experimental.pallas import tpu_sc as plsc

NE = 50000
NU = 20000
D = 64
R = 23
NC, NS = 2, 16
NW = NC * NS

E_REAL = 800000
E_PAD = 819200
EDGES_PER_SC_WORKER = E_PAD // NS
EB = 128
N_EBATCH = EDGES_PER_SC_WORKER // EB
HALF = D // 2
ENT_OUT_ROWS = 50176
ACC_ROWS = 50304
QROWS = 50176

NNZ_REAL = 500000
NNZ_PAD = 516096
NNZ_PER_WORKER = NNZ_PAD // NW
UB = 128
N_UBATCH = NNZ_PER_WORKER // UB
U_BLOCKS = 157
U_ROWS = U_BLOCKS * 128


def _q_body(x_ref, w_ref, o_ref):
    x = x_ref[...]
    w = w_ref[...]
    o_ref[...] = lax.dot_general(x * x, w * w, (((1,), (1,)), ((), ())),
                                 preferred_element_type=jnp.float32)


def _compute_q(emb_pad, wpad):
    blk = 1792
    return pl.pallas_call(
        _q_body,
        grid=(QROWS // blk,),
        in_specs=[
            pl.BlockSpec((blk, D), lambda i: (i, 0)),
            pl.BlockSpec((128, D), lambda i: (0, 0)),
        ],
        out_specs=pl.BlockSpec((blk, 128), lambda i: (i, 0)),
        out_shape=jax.ShapeDtypeStruct((QROWS, 128), jnp.float32),
    )(emb_pad, wpad)


def _entity_sc_body(qflat_hbm, emb2_hbm, head_hbm, tail_hbm, y_hbm, w2_hbm,
                    zr_hbm, zd_hbm,
                    num_hbm, den_hbm,
                    acc_sh, den_sh,
                    hbuf0, hbuf1, tbuf0, tbuf1, ybuf0, ybuf1,
                    qih0, qih1, qit0, qit1, qhv0, qhv1, qtv0, qtv1,
                    ewb0, ewb1, trows0, trows1, orows0, orows1,
                    sidx0, sidx1, kst0, kst1, tst0, tst1, wtab,
                    isem0, isem1, gsem0, gsem1, ssem0, ssem1):
    cid = lax.axis_index("c")
    sid = lax.axis_index("s")
    hbufs = (hbuf0, hbuf1)
    tbufs = (tbuf0, tbuf1)
    ybufs = (ybuf0, ybuf1)
    qihs = (qih0, qih1)
    qits = (qit0, qit1)
    qhvs = (qhv0, qhv1)
    qtvs = (qtv0, qtv1)
    ewbs = (ewb0, ewb1)
    trowss = (trows0, trows1)
    orowss = (orows0, orows1)
    sidxs = (sidx0, sidx1)
    ksts = (kst0, kst1)
    tsts = (tst0, tst1)
    isems = (isem0, isem1)
    gsems = (gsem0, gsem1)
    ssems = (ssem0, ssem1)

    pltpu.sync_copy(w2_hbm.at[cid], wtab)

    for j in range(25):
        b = j * 16 + sid
        @pl.when(b < ACC_ROWS // 128)
        def _():
            pltpu.sync_copy(zr_hbm, acc_sh.at[pl.ds(b * 128, 128)])
            pltpu.sync_copy(zd_hbm, den_sh.at[pl.ds(b * 128, 128)])
    plsc.subcore_barrier()

    def ebase(bi):
        return sid * EDGES_PER_SC_WORKER + bi * EB

    def issue_idx(bi, buf):
        off = ebase(bi)
        pltpu.async_copy(head_hbm.at[pl.ds(off, EB)], hbufs[buf], isems[buf])
        pltpu.async_copy(tail_hbm.at[pl.ds(off, EB)], tbufs[buf], isems[buf])
        pltpu.async_copy(y_hbm.at[pl.ds(off, EB)], ybufs[buf], isems[buf])

    def wait_idx(bi, buf):
        off = ebase(bi)
        pltpu.make_async_copy(head_hbm.at[pl.ds(off, EB)], hbufs[buf], isems[buf]).wait()
        pltpu.make_async_copy(tail_hbm.at[pl.ds(off, EB)], tbufs[buf], isems[buf]).wait()
        pltpu.make_async_copy(y_hbm.at[pl.ds(off, EB)], ybufs[buf], isems[buf]).wait()

    def issue_gather(buf):
        pltpu.async_copy(qflat_hbm.at[qihs[buf]], qhvs[buf], gsems[buf])
        pltpu.async_copy(qflat_hbm.at[qits[buf]], qtvs[buf], gsems[buf])
        pltpu.async_copy(emb2_hbm.at[cid].at[tsts[buf]], trowss[buf], gsems[buf])

    def wait_gather(buf):
        pltpu.make_async_copy(qflat_hbm.at[qihs[buf]], qhvs[buf], gsems[buf]).wait()
        pltpu.make_async_copy(qflat_hbm.at[qits[buf]], qtvs[buf], gsems[buf]).wait()
        pltpu.make_async_copy(emb2_hbm.at[cid].at[tsts[buf]], trowss[buf], gsems[buf]).wait()

    def issue_scatter(buf):
        pltpu.async_copy(orowss[buf], acc_sh.at[sidxs[buf]], ssems[buf], add=True)
        pltpu.async_copy(ewbs[buf], den_sh.at[sidxs[buf]], ssems[buf], add=True)

    def wait_scatter(buf):
        pltpu.make_async_copy(orowss[buf], acc_sh.at[sidxs[buf]], ssems[buf]).wait()
        pltpu.make_async_copy(ewbs[buf], den_sh.at[sidxs[buf]], ssems[buf]).wait()

    def prep_indices(buf):
        for c in range(EB // 16):
            s = pl.ds(c * 16, 16)
            h = hbufs[buf][s]
            t = tbufs[buf][s]
            y = ybufs[buf][s]
            k = jnp.where(y == 0, 22, y - 1)
            ksts[buf][s] = k
            qihs[buf][s] = h * 128 + k
            qits[buf][s] = t * 128 + k
            sidxs[buf][s] = h
            tsts[buf][s] = t

    def compute_batch(buf):
        for c in range(EB // 16):
            s = pl.ds(c * 16, 16)
            ew = jnp.exp(qhvs[buf][s] * qtvs[buf][s])
            ewbs[buf][s] = ew
            kc = ksts[buf][s]
            for l in range(16):
                e = c * 16 + l
                ewv = jnp.full((16,), ew[l], jnp.float32)
                kbase = kc[l] * HALF
                for j in range(HALF // 16):
                    tj = trowss[buf][e, pl.ds(j * 16, 16)]
                    rj = wtab[pl.ds(kbase + j * 16, 16)]
                    orowss[buf][e, pl.ds(j * 16, 16)] = ewv * (tj * rj)

    issue_idx(0, 0)
    issue_idx(1, 1)

    def half_iter(bi, buf):
        obuf = 1 - buf
        wait_idx(bi, buf)
        @pl.when(bi >= 2)
        def _():
            wait_scatter(buf)
        prep_indices(buf)
        issue_gather(buf)
        @pl.when(bi + 2 < N_EBATCH)
        def _():
            issue_idx(bi + 2, buf)
        @pl.when(bi >= 1)
        def _():
            wait_gather(obuf)
            compute_batch(obuf)
            issue_scatter(obuf)

    def loop_body(i, carry):
        half_iter(2 * i, 0)
        half_iter(2 * i + 1, 1)
        return carry

    lax.fori_loop(0, N_EBATCH // 2, loop_body, 0)

    wait_gather(1)
    compute_batch(1)
    issue_scatter(1)
    wait_scatter(0)
    wait_scatter(1)
    plsc.subcore_barrier()

    for jb in range(25):
        b = jb * 16 + sid
        @pl.when(b < ENT_OUT_ROWS // 128)
        def _():
            pltpu.sync_copy(acc_sh.at[pl.ds(b * 128, 128)],
                            num_hbm.at[cid].at[pl.ds(b * 128, 128)])
            pltpu.sync_copy(den_sh.at[pl.ds(b * 128, 128)],
                            den_hbm.at[cid].at[pl.ds(b * 128, 128)])


def _entity_sc(qflat, emb2, head_p, tail_p, y_p, w2, zr, zd):
    mesh = plsc.VectorSubcoreMesh(core_axis_name="c", subcore_axis_name="s",
                                  num_cores=NC, num_subcores=NS)
    ib = lambda: pltpu.VMEM((EB,), jnp.int32)
    fb = lambda: pltpu.VMEM((EB,), jnp.float32)
    f = pl.kernel(
        _entity_sc_body,
        out_type=(
            jax.ShapeDtypeStruct((NC, ENT_OUT_ROWS, HALF), jnp.float32),
            jax.ShapeDtypeStruct((NC, ENT_OUT_ROWS), jnp.float32),
        ),
        mesh=mesh,
        compiler_params=pltpu.CompilerParams(use_tc_tiling_on_sc=False),
        scratch_types=[
            pltpu.VMEM_SHARED((ACC_ROWS, HALF), jnp.float32),
            pltpu.VMEM_SHARED((ACC_ROWS,), jnp.float32),
            ib(), ib(), ib(), ib(), ib(), ib(),
            ib(), ib(), ib(), ib(),
            fb(), fb(), fb(), fb(),
            fb(), fb(),
            pltpu.VMEM((EB, HALF), jnp.float32),
            pltpu.VMEM((EB, HALF), jnp.float32),
            pltpu.VMEM((EB, HALF), jnp.float32),
            pltpu.VMEM((EB, HALF), jnp.float32),
            ib(), ib(), ib(), ib(), ib(), ib(),
            pltpu.VMEM((R * HALF,), jnp.float32),
            pltpu.SemaphoreType.DMA, pltpu.SemaphoreType.DMA,
            pltpu.SemaphoreType.DMA, pltpu.SemaphoreType.DMA,
            pltpu.SemaphoreType.DMA, pltpu.SemaphoreType.DMA,
        ],
    )
    return f(qflat, emb2, head_p, tail_p, y_p, w2, zr, zd)


def _div_body(num_ref, den_ref, o_ref):
    num = num_ref[...]
    den = den_ref[...]
    inv0 = (1.0 / jnp.maximum(den[0], 1e-37))[:, None]
    inv1 = (1.0 / jnp.maximum(den[1], 1e-37))[:, None]
    o_ref[...] = jnp.concatenate([num[0] * inv0, num[1] * inv1], axis=1)


def _entity_div(num, den):
    blk = 1792
    return pl.pallas_call(
        _div_body,
        grid=(ENT_OUT_ROWS // blk,),
        in_specs=[
            pl.BlockSpec((NC, blk, HALF), lambda i: (0, i, 0)),
            pl.BlockSpec((NC, blk), lambda i: (0, i)),
        ],
        out_specs=pl.BlockSpec((blk, D), lambda i: (i, 0)),
        out_shape=jax.ShapeDtypeStruct((ENT_OUT_ROWS, D), jnp.float32),
    )(num, den)


def _user_sc_body(emb_hbm, cols_hbm, rows_hbm, vals_hbm, zu_hbm, out_hbm,
                  uacc_sh,
                  cbuf0, cbuf1, rbuf0, rbuf1, vbuf0, vbuf1,
                  sidx0, sidx1, vst0, vst1, cst0, cst1,
                  erows0, erows1, orows0, orows1,
                  isem0, isem1, gsem0, gsem1, ssem0, ssem1):
    cid = lax.axis_index("c")
    sid = lax.axis_index("s")
    wid = sid * NC + cid
    cbufs = (cbuf0, cbuf1)
    rbufs = (rbuf0, rbuf1)
    vbufs = (vbuf0, vbuf1)
    sidxs = (sidx0, sidx1)
    vsts = (vst0, vst1)
    csts = (cst0, cst1)
    erowss = (erows0, erows1)
    orowss = (orows0, orows1)
    isems = (isem0, isem1)
    gsems = (gsem0, gsem1)
    ssems = (ssem0, ssem1)

    for j in range(10):
        b = j * 16 + sid
        @pl.when(b < U_BLOCKS)
        def _():
            pltpu.sync_copy(zu_hbm, uacc_sh.at[pl.ds(b * 128, 128)])
    plsc.subcore_barrier()

    def ubase(bi):
        return wid * NNZ_PER_WORKER + bi * UB

    def issue_idx(bi, buf):
        off = ubase(bi)
        pltpu.async_copy(cols_hbm.at[pl.ds(off, UB)], cbufs[buf], isems[buf])
        pltpu.async_copy(rows_hbm.at[pl.ds(off, UB)], rbufs[buf], isems[buf])
        pltpu.async_copy(vals_hbm.at[pl.ds(off, UB)], vbufs[buf], isems[buf])

    def wait_idx(bi, buf):
        off = ubase(bi)
        pltpu.make_async_copy(cols_hbm.at[pl.ds(off, UB)], cbufs[buf], isems[buf]).wait()
        pltpu.make_async_copy(rows_hbm.at[pl.ds(off, UB)], rbufs[buf], isems[buf]).wait()
        pltpu.make_async_copy(vals_hbm.at[pl.ds(off, UB)], vbufs[buf], isems[buf]).wait()

    def issue_gather(buf):
        pltpu.async_copy(emb_hbm.at[csts[buf]], erowss[buf], gsems[buf])

    def wait_gather(buf):
        pltpu.make_async_copy(emb_hbm.at[csts[buf]], erowss[buf], gsems[buf]).wait()

    def issue_scatter(buf):
        pltpu.async_copy(orowss[buf], uacc_sh.at[sidxs[buf]], ssems[buf], add=True)

    def wait_scatter(buf):
        pltpu.make_async_copy(orowss[buf], uacc_sh.at[sidxs[buf]], ssems[buf]).wait()

    def stash(buf):
        for c in range(UB // 16):
            s = pl.ds(c * 16, 16)
            sidxs[buf][s] = rbufs[buf][s]
            vsts[buf][s] = vbufs[buf][s]
            csts[buf][s] = cbufs[buf][s]

    def compute_batch(buf):
        for c in range(UB // 16):
            vc = vsts[buf][pl.ds(c * 16, 16)]
            for l in range(16):
                e = c * 16 + l
                vv = jnp.full((16,), vc[l], jnp.float32)
                for j in range(D // 16):
                    orowss[buf][e, pl.ds(j * 16, 16)] = (
                        vv * erowss[buf][e, pl.ds(j * 16, 16)])

    issue_idx(0, 0)
    issue_idx(1, 1)

    def half_iter(bi, buf):
        obuf = 1 - buf
        wait_idx(bi, buf)
        @pl.when(bi >= 2)
        def _():
            wait_scatter(buf)
        stash(buf)
        issue_gather(buf)
        @pl.when(bi + 2 < N_UBATCH)
        def _():
            issue_idx(bi + 2, buf)
        @pl.when(bi >= 1)
        def _():
            wait_gather(obuf)
            compute_batch(obuf)
            issue_scatter(obuf)

    def loop_body(i, carry):
        half_iter(2 * i, 0)
        half_iter(2 * i + 1, 1)
        return carry

    lax.fori_loop(0, N_UBATCH // 2, loop_body, 0)

    wait_gather(1)
    compute_batch(1)
    issue_scatter(1)
    wait_scatter(0)
    wait_scatter(1)
    plsc.subcore_barrier()

    for j in range(10):
        b = j * 16 + sid
        @pl.when(b < U_BLOCKS)
        def _():
            pltpu.sync_copy(uacc_sh.at[pl.ds(b * 128, 128)],
                            out_hbm.at[cid].at[pl.ds(b * 128, 128)])


def _user_sc(emb, cols_p, rows_p, vals_p, zu):
    mesh = plsc.VectorSubcoreMesh(core_axis_name="c", subcore_axis_name="s",
                                  num_cores=NC, num_subcores=NS)
    ib = lambda: pltpu.VMEM((UB,), jnp.int32)
    fb = lambda: pltpu.VMEM((UB,), jnp.float32)
    f = pl.kernel(
        _user_sc_body,
        out_type=jax.ShapeDtypeStruct((NC, U_ROWS, D), jnp.float32),
        mesh=mesh,
        compiler_params=pltpu.CompilerParams(use_tc_tiling_on_sc=False),
        scratch_types=[
            pltpu.VMEM_SHARED((U_ROWS, D), jnp.float32),
            ib(), ib(), ib(), ib(), fb(), fb(),
            ib(), ib(), fb(), fb(), ib(), ib(),
            pltpu.VMEM((UB, D), jnp.float32),
            pltpu.VMEM((UB, D), jnp.float32),
            pltpu.VMEM((UB, D), jnp.float32),
            pltpu.VMEM((UB, D), jnp.float32),
            pltpu.SemaphoreType.DMA, pltpu.SemaphoreType.DMA,
            pltpu.SemaphoreType.DMA, pltpu.SemaphoreType.DMA,
            pltpu.SemaphoreType.DMA, pltpu.SemaphoreType.DMA,
        ],
    )
    return f(emb, cols_p, rows_p, vals_p, zu)


def _ep_body(ue_ref, w_ref, p_ref, o_ref):
    ue = ue_ref[...]
    w = w_ref[...]
    logits = lax.dot_general(ue, w, (((1,), (1,)), ((), ())),
                             preferred_element_type=jnp.float32)
    col = lax.broadcasted_iota(jnp.int32, logits.shape, 1)
    logits = jnp.where(col < R, logits, -1e30)
    m = jnp.max(logits, axis=-1, keepdims=True)
    ex = jnp.exp(logits - m)
    score = ex / jnp.sum(ex, axis=-1, keepdims=True)
    mult = jnp.dot(score, w, preferred_element_type=jnp.float32)
    p = p_ref[0] + p_ref[1]
    o_ref[...] = p * (1.0 + mult)


def _epilogue(user_emb, w24, partials):
    blk = 1000
    return pl.pallas_call(
        _ep_body,
        grid=(NU // blk,),
        in_specs=[
            pl.BlockSpec((blk, D), lambda i: (i, 0)),
            pl.BlockSpec((24, D), lambda i: (0, 0)),
            pl.BlockSpec((NC, blk, D), lambda i: (0, i, 0)),
        ],
        out_specs=pl.BlockSpec((blk, D), lambda i: (i, 0)),
        out_shape=jax.ShapeDtypeStruct((NU, D), jnp.float32),
    )(user_emb, w24, partials)


def kernel(entity_emb, user_emb, edge_index, edge_type, interact_rows,
           interact_cols, interact_values, weight):
    i32 = jnp.int32
    head = edge_index[0].astype(i32)
    tail = edge_index[1].astype(i32)
    etype = edge_type.astype(i32)

    emb_pad = jnp.pad(entity_emb, ((0, QROWS - NE), (0, 0)))
    wpad = jnp.pad(weight, ((0, 128 - R), (0, 0)))
    q = _compute_q(emb_pad, wpad)
    qflat = q.reshape(-1)

    npad = E_PAD - E_REAL
    head_p = jnp.concatenate([head, jnp.full((npad,), NE, i32)])
    tail_p = jnp.concatenate([tail, jnp.zeros((npad,), i32)])
    y_p = jnp.concatenate([etype, jnp.ones((npad,), i32)])

    emb2 = jnp.stack([entity_emb[:, :HALF], entity_emb[:, HALF:]])
    w2 = jnp.stack([weight[:, :HALF].reshape(-1), weight[:, HALF:].reshape(-1)])

    zr = jnp.zeros((128, HALF), jnp.float32)
    zd = jnp.zeros((128,), jnp.float32)
    num, den = _entity_sc(qflat, emb2, head_p, tail_p, y_p, w2, zr, zd)
    entity_agg = _entity_div(num, den)[:NE]

    upad = NNZ_PAD - NNZ_REAL
    cols_p = jnp.concatenate([interact_cols.astype(i32), jnp.zeros((upad,), i32)])
    rows_p = jnp.concatenate([interact_rows.astype(i32), jnp.zeros((upad,), i32)])
    vals_p = jnp.concatenate([interact_values, jnp.zeros((upad,), jnp.float32)])
    zu = jnp.zeros((128, D), jnp.float32)
    partials = _user_sc(entity_emb, cols_p, rows_p, vals_p, zu)

    w24 = jnp.pad(weight, ((0, 1), (0, 0)))
    user_agg = _epilogue(user_emb, w24, partials)
    return (entity_agg, user_agg)


# speedup vs baseline: 9.6899x; 1.1164x over previous
import functools

import jax
import jax.numpy as jnp
from jax import lax
from jax.experimental import pallas as pl
from jax.experimental.pallas import tpu as pltpu
from jax.experimental.pallas import tpu_sc as plsc

NE = 50000
NU = 20000
D = 64
R = 23
NC, NS = 2, 16
NW = NC * NS

E_REAL = 800000
E_PAD = 824832
EDGES_PER_SC_WORKER = E_PAD // NS
EB = 96
N_EBATCH = EDGES_PER_SC_WORKER // EB
HALF = D // 2
ENT_OUT_ROWS = 50176
ACC_ROWS = 50176
QROWS = 50176

NNZ_REAL = 500000
NNZ_PAD = 506880
NNZ_PER_WORKER = NNZ_PAD // NW
UB = 96
N_UBATCH = NNZ_PER_WORKER // UB
U_BLOCKS = 157
U_ROWS = U_BLOCKS * 128


def _q_body(x_ref, w_ref, o_ref):
    x = x_ref[...]
    w = w_ref[...]
    o_ref[...] = lax.dot_general(x * x, w * w, (((1,), (1,)), ((), ())),
                                 preferred_element_type=jnp.float32)


def _compute_q(emb_pad, wpad):
    blk = 1792
    return pl.pallas_call(
        _q_body,
        grid=(QROWS // blk,),
        in_specs=[
            pl.BlockSpec((blk, D), lambda i: (i, 0)),
            pl.BlockSpec((128, D), lambda i: (0, 0)),
        ],
        out_specs=pl.BlockSpec((blk, 128), lambda i: (i, 0)),
        out_shape=jax.ShapeDtypeStruct((QROWS, 128), jnp.float32),
    )(emb_pad, wpad)


def _entity_sc_body(*refs):
    (qflat_hbm, emb2_hbm, head_hbm, tail_hbm, y_hbm, w2_hbm, zr_hbm, zd_hbm,
     num_hbm, den_hbm, acc_sh, den_sh) = refs[:12]
    r = refs[12:]
    groups = [tuple(r[i * 3:(i + 1) * 3]) for i in range(13)]
    (hbufs, tbufs, ybufs, qihs, qits, qhvs, qtvs, ewbs,
     trowss, orowss, sidxs, ksts, tsts) = groups
    wtab = r[39]
    isems = r[40:43]
    gsems = r[43:46]
    ssems = r[46:49]

    cid = lax.axis_index("c")
    sid = lax.axis_index("s")

    pltpu.sync_copy(w2_hbm.at[cid], wtab)

    for j in range(25):
        b = j * 16 + sid
        @pl.when(b < ACC_ROWS // 128)
        def _():
            pltpu.sync_copy(zr_hbm, acc_sh.at[pl.ds(b * 128, 128)])
            pltpu.sync_copy(zd_hbm, den_sh.at[pl.ds(b * 128, 128)])
    plsc.subcore_barrier()

    def ebase(bi):
        return sid * EDGES_PER_SC_WORKER + bi * EB

    def issue_idx(bi, sl):
        off = ebase(bi)
        pltpu.async_copy(head_hbm.at[pl.ds(off, EB)], hbufs[sl], isems[sl])
        pltpu.async_copy(tail_hbm.at[pl.ds(off, EB)], tbufs[sl], isems[sl])
        pltpu.async_copy(y_hbm.at[pl.ds(off, EB)], ybufs[sl], isems[sl])

    def wait_idx(bi, sl):
        off = ebase(bi)
        pltpu.make_async_copy(head_hbm.at[pl.ds(off, EB)], hbufs[sl], isems[sl]).wait()
        pltpu.make_async_copy(tail_hbm.at[pl.ds(off, EB)], tbufs[sl], isems[sl]).wait()
        pltpu.make_async_copy(y_hbm.at[pl.ds(off, EB)], ybufs[sl], isems[sl]).wait()

    def issue_gather(sl):
        pltpu.async_copy(qflat_hbm.at[qihs[sl]], qhvs[sl], gsems[sl])
        pltpu.async_copy(qflat_hbm.at[qits[sl]], qtvs[sl], gsems[sl])
        pltpu.async_copy(emb2_hbm.at[cid].at[tsts[sl]], trowss[sl], gsems[sl])

    def wait_gather(sl):
        pltpu.make_async_copy(qflat_hbm.at[qihs[sl]], qhvs[sl], gsems[sl]).wait()
        pltpu.make_async_copy(qflat_hbm.at[qits[sl]], qtvs[sl], gsems[sl]).wait()
        pltpu.make_async_copy(emb2_hbm.at[cid].at[tsts[sl]], trowss[sl], gsems[sl]).wait()

    def issue_scatter(sl):
        pltpu.async_copy(orowss[sl], acc_sh.at[sidxs[sl]], ssems[sl], add=True)
        pltpu.async_copy(ewbs[sl], den_sh.at[sidxs[sl]], ssems[sl], add=True)

    def wait_scatter(sl):
        pltpu.make_async_copy(orowss[sl], acc_sh.at[sidxs[sl]], ssems[sl]).wait()
        pltpu.make_async_copy(ewbs[sl], den_sh.at[sidxs[sl]], ssems[sl]).wait()

    def prep_indices(sl):
        for c in range(EB // 16):
            s = pl.ds(c * 16, 16)
            h = hbufs[sl][s]
            t = tbufs[sl][s]
            y = ybufs[sl][s]
            k = jnp.where(y == 0, 22, y - 1)
            ksts[sl][s] = k
            qihs[sl][s] = h * 128 + k
            qits[sl][s] = t * 128 + k
            sidxs[sl][s] = h
            tsts[sl][s] = t

    def compute_batch(sl):
        for c in range(EB // 16):
            s = pl.ds(c * 16, 16)
            ew = jnp.exp(qhvs[sl][s] * qtvs[sl][s])
            ewbs[sl][s] = ew
            kc = ksts[sl][s]
            for l in range(16):
                e = c * 16 + l
                ewv = jnp.full((16,), ew[l], jnp.float32)
                kbase = kc[l] * HALF
                for j in range(HALF // 16):
                    tj = trowss[sl][e, pl.ds(j * 16, 16)]
                    rj = wtab[pl.ds(kbase + j * 16, 16)]
                    orowss[sl][e, pl.ds(j * 16, 16)] = ewv * (tj * rj)

    issue_idx(0, 0)
    issue_idx(1, 1)
    issue_idx(2, 2)

    NB = N_EBATCH

    def loop_body(i, carry):
        for b in range(3):
            bi = 3 * i + b
            sl = b
            cl = (b + 1) % 3

            @pl.when(bi < NB)
            def _():
                wait_idx(bi, sl)
            @pl.when(bi >= 3)
            def _():
                wait_scatter(sl)
            @pl.when(bi < NB)
            def _():
                prep_indices(sl)
                issue_gather(sl)
            @pl.when(bi + 3 < NB)
            def _():
                issue_idx(bi + 3, sl)
            @pl.when(jnp.logical_and(bi >= 2, bi <= NB + 1))
            def _():
                wait_gather(cl)
                compute_batch(cl)
                issue_scatter(cl)
        return carry

    lax.fori_loop(0, (NB + 3) // 3, loop_body, 0)
    plsc.subcore_barrier()

    for jb in range(25):
        b = jb * 16 + sid
        @pl.when(b < ENT_OUT_ROWS // 128)
        def _():
            pltpu.sync_copy(acc_sh.at[pl.ds(b * 128, 128)],
                            num_hbm.at[cid].at[pl.ds(b * 128, 128)])
            pltpu.sync_copy(den_sh.at[pl.ds(b * 128, 128)],
                            den_hbm.at[cid].at[pl.ds(b * 128, 128)])


def _entity_sc(qflat, emb2, head_p, tail_p, y_p, w2, zr, zd):
    mesh = plsc.VectorSubcoreMesh(core_axis_name="c", subcore_axis_name="s",
                                  num_cores=NC, num_subcores=NS)
    ib = lambda: pltpu.VMEM((EB,), jnp.int32)
    fb = lambda: pltpu.VMEM((EB,), jnp.float32)
    rb = lambda: pltpu.VMEM((EB, HALF), jnp.float32)
    scratch = [
        pltpu.VMEM_SHARED((ACC_ROWS, HALF), jnp.float32),
        pltpu.VMEM_SHARED((ACC_ROWS,), jnp.float32),
    ]
    scratch += [ib() for _ in range(9)]
    scratch += [ib() for _ in range(6)]
    scratch += [fb() for _ in range(6)]
    scratch += [fb() for _ in range(3)]
    scratch += [rb() for _ in range(6)]
    scratch += [ib() for _ in range(9)]
    scratch += [pltpu.VMEM((R * HALF,), jnp.float32)]
    scratch += [pltpu.SemaphoreType.DMA for _ in range(9)]
    f = pl.kernel(
        _entity_sc_body,
        out_type=(
            jax.ShapeDtypeStruct((NC, ENT_OUT_ROWS, HALF), jnp.float32),
            jax.ShapeDtypeStruct((NC, ENT_OUT_ROWS), jnp.float32),
        ),
        mesh=mesh,
        compiler_params=pltpu.CompilerParams(use_tc_tiling_on_sc=False),
        scratch_types=scratch,
    )
    return f(qflat, emb2, head_p, tail_p, y_p, w2, zr, zd)


def _div_body(num_ref, den_ref, o_ref):
    num = num_ref[...]
    den = den_ref[...]
    inv0 = (1.0 / jnp.maximum(den[0], 1e-37))[:, None]
    inv1 = (1.0 / jnp.maximum(den[1], 1e-37))[:, None]
    o_ref[...] = jnp.concatenate([num[0] * inv0, num[1] * inv1], axis=1)


def _entity_div(num, den):
    blk = 1792
    return pl.pallas_call(
        _div_body,
        grid=(ENT_OUT_ROWS // blk,),
        in_specs=[
            pl.BlockSpec((NC, blk, HALF), lambda i: (0, i, 0)),
            pl.BlockSpec((NC, blk), lambda i: (0, i)),
        ],
        out_specs=pl.BlockSpec((blk, D), lambda i: (i, 0)),
        out_shape=jax.ShapeDtypeStruct((ENT_OUT_ROWS, D), jnp.float32),
    )(num, den)


def _user_sc_body(*refs):
    (emb_hbm, cols_hbm, rows_hbm, vals_hbm, zu_hbm, out_hbm, uacc_sh) = refs[:7]
    r = refs[7:]
    groups = [tuple(r[i * 3:(i + 1) * 3]) for i in range(8)]
    (cbufs, rbufs, vbufs, sidxs, vsts, csts, erowss, orowss) = groups
    isems = r[24:27]
    gsems = r[27:30]
    ssems = r[30:33]

    cid = lax.axis_index("c")
    sid = lax.axis_index("s")
    wid = sid * NC + cid

    for j in range(10):
        b = j * 16 + sid
        @pl.when(b < U_BLOCKS)
        def _():
            pltpu.sync_copy(zu_hbm, uacc_sh.at[pl.ds(b * 128, 128)])
    plsc.subcore_barrier()

    def ubase(bi):
        return wid * NNZ_PER_WORKER + bi * UB

    def issue_idx(bi, sl):
        off = ubase(bi)
        pltpu.async_copy(cols_hbm.at[pl.ds(off, UB)], cbufs[sl], isems[sl])
        pltpu.async_copy(rows_hbm.at[pl.ds(off, UB)], rbufs[sl], isems[sl])
        pltpu.async_copy(vals_hbm.at[pl.ds(off, UB)], vbufs[sl], isems[sl])

    def wait_idx(bi, sl):
        off = ubase(bi)
        pltpu.make_async_copy(cols_hbm.at[pl.ds(off, UB)], cbufs[sl], isems[sl]).wait()
        pltpu.make_async_copy(rows_hbm.at[pl.ds(off, UB)], rbufs[sl], isems[sl]).wait()
        pltpu.make_async_copy(vals_hbm.at[pl.ds(off, UB)], vbufs[sl], isems[sl]).wait()

    def issue_gather(sl):
        pltpu.async_copy(emb_hbm.at[csts[sl]], erowss[sl], gsems[sl])

    def wait_gather(sl):
        pltpu.make_async_copy(emb_hbm.at[csts[sl]], erowss[sl], gsems[sl]).wait()

    def issue_scatter(sl):
        pltpu.async_copy(orowss[sl], uacc_sh.at[sidxs[sl]], ssems[sl], add=True)

    def wait_scatter(sl):
        pltpu.make_async_copy(orowss[sl], uacc_sh.at[sidxs[sl]], ssems[sl]).wait()

    def stash(sl):
        for c in range(UB // 16):
            s = pl.ds(c * 16, 16)
            sidxs[sl][s] = rbufs[sl][s]
            vsts[sl][s] = vbufs[sl][s]
            csts[sl][s] = cbufs[sl][s]

    def compute_batch(sl):
        for c in range(UB // 16):
            vc = vsts[sl][pl.ds(c * 16, 16)]
            for l in range(16):
                e = c * 16 + l
                vv = jnp.full((16,), vc[l], jnp.float32)
                for j in range(D // 16):
                    orowss[sl][e, pl.ds(j * 16, 16)] = (
                        vv * erowss[sl][e, pl.ds(j * 16, 16)])

    issue_idx(0, 0)
    issue_idx(1, 1)
    issue_idx(2, 2)

    NB = N_UBATCH

    def loop_body(i, carry):
        for b in range(3):
            bi = 3 * i + b
            sl = b
            cl = (b + 1) % 3

            @pl.when(bi < NB)
            def _():
                wait_idx(bi, sl)
            @pl.when(bi >= 3)
            def _():
                wait_scatter(sl)
            @pl.when(bi < NB)
            def _():
                stash(sl)
                issue_gather(sl)
            @pl.when(bi + 3 < NB)
            def _():
                issue_idx(bi + 3, sl)
            @pl.when(jnp.logical_and(bi >= 2, bi <= NB + 1))
            def _():
                wait_gather(cl)
                compute_batch(cl)
                issue_scatter(cl)
        return carry

    lax.fori_loop(0, (NB + 3) // 3, loop_body, 0)
    plsc.subcore_barrier()

    for j in range(10):
        b = j * 16 + sid
        @pl.when(b < U_BLOCKS)
        def _():
            pltpu.sync_copy(uacc_sh.at[pl.ds(b * 128, 128)],
                            out_hbm.at[cid].at[pl.ds(b * 128, 128)])


def _user_sc(emb, cols_p, rows_p, vals_p, zu):
    mesh = plsc.VectorSubcoreMesh(core_axis_name="c", subcore_axis_name="s",
                                  num_cores=NC, num_subcores=NS)
    ib = lambda: pltpu.VMEM((UB,), jnp.int32)
    fb = lambda: pltpu.VMEM((UB,), jnp.float32)
    db = lambda: pltpu.VMEM((UB, D), jnp.float32)
    scratch = [pltpu.VMEM_SHARED((U_ROWS, D), jnp.float32)]
    scratch += [ib() for _ in range(6)]
    scratch += [fb() for _ in range(3)]
    scratch += [ib() for _ in range(3)]
    scratch += [fb() for _ in range(3)]
    scratch += [ib() for _ in range(3)]
    scratch += [db() for _ in range(6)]
    scratch += [pltpu.SemaphoreType.DMA for _ in range(9)]
    f = pl.kernel(
        _user_sc_body,
        out_type=jax.ShapeDtypeStruct((NC, U_ROWS, D), jnp.float32),
        mesh=mesh,
        compiler_params=pltpu.CompilerParams(use_tc_tiling_on_sc=False),
        scratch_types=scratch,
    )
    return f(emb, cols_p, rows_p, vals_p, zu)


def _ep_body(ue_ref, w_ref, p_ref, o_ref):
    ue = ue_ref[...]
    w = w_ref[...]
    logits = lax.dot_general(ue, w, (((1,), (1,)), ((), ())),
                             preferred_element_type=jnp.float32)
    col = lax.broadcasted_iota(jnp.int32, logits.shape, 1)
    logits = jnp.where(col < R, logits, -1e30)
    m = jnp.max(logits, axis=-1, keepdims=True)
    ex = jnp.exp(logits - m)
    score = ex / jnp.sum(ex, axis=-1, keepdims=True)
    mult = jnp.dot(score, w, preferred_element_type=jnp.float32)
    p = p_ref[0] + p_ref[1]
    o_ref[...] = p * (1.0 + mult)


def _epilogue(user_emb, w24, partials):
    blk = 1000
    return pl.pallas_call(
        _ep_body,
        grid=(NU // blk,),
        in_specs=[
            pl.BlockSpec((blk, D), lambda i: (i, 0)),
            pl.BlockSpec((24, D), lambda i: (0, 0)),
            pl.BlockSpec((NC, blk, D), lambda i: (0, i, 0)),
        ],
        out_specs=pl.BlockSpec((blk, D), lambda i: (i, 0)),
        out_shape=jax.ShapeDtypeStruct((NU, D), jnp.float32),
    )(user_emb, w24, partials)


def kernel(entity_emb, user_emb, edge_index, edge_type, interact_rows,
           interact_cols, interact_values, weight):
    i32 = jnp.int32
    head = edge_index[0].astype(i32)
    tail = edge_index[1].astype(i32)
    etype = edge_type.astype(i32)

    emb_pad = jnp.pad(entity_emb, ((0, QROWS - NE), (0, 0)))
    wpad = jnp.pad(weight, ((0, 128 - R), (0, 0)))
    q = _compute_q(emb_pad, wpad)
    qflat = q.reshape(-1)

    npad = E_PAD - E_REAL
    head_p = jnp.concatenate([head, jnp.full((npad,), NE, i32)])
    tail_p = jnp.concatenate([tail, jnp.zeros((npad,), i32)])
    y_p = jnp.concatenate([etype, jnp.ones((npad,), i32)])

    emb2 = jnp.stack([entity_emb[:, :HALF], entity_emb[:, HALF:]])
    w2 = jnp.stack([weight[:, :HALF].reshape(-1), weight[:, HALF:].reshape(-1)])

    zr = jnp.zeros((128, HALF), jnp.float32)
    zd = jnp.zeros((128,), jnp.float32)
    num, den = _entity_sc(qflat, emb2, head_p, tail_p, y_p, w2, zr, zd)
    entity_agg = _entity_div(num, den)[:NE]

    upad = NNZ_PAD - NNZ_REAL
    cols_p = jnp.concatenate([interact_cols.astype(i32), jnp.zeros((upad,), i32)])
    rows_p = jnp.concatenate([interact_rows.astype(i32), jnp.zeros((upad,), i32)])
    vals_p = jnp.concatenate([interact_values, jnp.zeros((upad,), jnp.float32)])
    zu = jnp.zeros((128, D), jnp.float32)
    partials = _user_sc(entity_emb, cols_p, rows_p, vals_p, zu)

    w24 = jnp.pad(weight, ((0, 1), (0, 0)))
    user_agg = _epilogue(user_emb, w24, partials)
    return (entity_agg, user_agg)


# speedup vs baseline: 9.8137x; 1.0128x over previous
import functools

import jax
import jax.numpy as jnp
from jax import lax
from jax.experimental import pallas as pl
from jax.experimental.pallas import tpu as pltpu
from jax.experimental.pallas import tpu_sc as plsc

NE = 50000
NU = 20000
D = 64
R = 23
NC, NS = 2, 16
NW = NC * NS

E_REAL = 800000
E_PAD = 801024
EDGES_PER_SC_WORKER = E_PAD // NS
EB = 112
N_EBATCH = EDGES_PER_SC_WORKER // EB
HALF = D // 2
ENT_OUT_ROWS = 50176
ACC_ROWS = 50176
QROWS = 50176

NNZ_REAL = 500000
NNZ_PAD = 505344
NNZ_PER_WORKER = NNZ_PAD // NW
UB = 112
N_UBATCH = NNZ_PER_WORKER // UB
U_BLOCKS = 157
U_ROWS = U_BLOCKS * 128


def _q_body(x_ref, w_ref, o_ref):
    x = x_ref[...]
    w = w_ref[...]
    o_ref[...] = lax.dot_general(x * x, w * w, (((1,), (1,)), ((), ())),
                                 preferred_element_type=jnp.float32)


def _compute_q(emb_pad, wpad):
    blk = 1792
    return pl.pallas_call(
        _q_body,
        grid=(QROWS // blk,),
        in_specs=[
            pl.BlockSpec((blk, D), lambda i: (i, 0)),
            pl.BlockSpec((128, D), lambda i: (0, 0)),
        ],
        out_specs=pl.BlockSpec((blk, 128), lambda i: (i, 0)),
        out_shape=jax.ShapeDtypeStruct((QROWS, 128), jnp.float32),
    )(emb_pad, wpad)


def _entity_sc_body(*refs):
    (qflat_hbm, emb2_hbm, edges_hbm, w2_hbm, zr_hbm, zd_hbm,
     num_hbm, den_hbm, acc_sh, den_sh) = refs[:10]
    r = refs[10:]
    groups = [tuple(r[i * 3:(i + 1) * 3]) for i in range(8)]
    (pbufs, qi2s, qv2s, ewbs, trowss, orowss, sidxs, ksts) = groups
    tsts = tuple(r[24:27])
    wtab = r[27]
    isems = r[28:31]
    gsems = r[31:34]
    ssems = r[34:37]

    cid = lax.axis_index("c")
    sid = lax.axis_index("s")

    pltpu.sync_copy(w2_hbm.at[cid], wtab)

    for j in range(25):
        b = j * 16 + sid
        @pl.when(b < ACC_ROWS // 128)
        def _():
            pltpu.sync_copy(zr_hbm, acc_sh.at[pl.ds(b * 128, 128)])
            pltpu.sync_copy(zd_hbm, den_sh.at[pl.ds(b * 128, 128)])
    plsc.subcore_barrier()

    def issue_idx(bi, sl):
        pltpu.async_copy(edges_hbm.at[sid].at[pl.ds(bi * 3 * EB, 3 * EB)],
                         pbufs[sl], isems[sl])

    def wait_idx(bi, sl):
        pltpu.make_async_copy(edges_hbm.at[sid].at[pl.ds(bi * 3 * EB, 3 * EB)],
                              pbufs[sl], isems[sl]).wait()

    def issue_gather(sl):
        pltpu.async_copy(qflat_hbm.at[qi2s[sl]], qv2s[sl], gsems[sl])
        pltpu.async_copy(emb2_hbm.at[cid].at[tsts[sl]], trowss[sl], gsems[sl])

    def wait_gather(sl):
        pltpu.make_async_copy(qflat_hbm.at[qi2s[sl]], qv2s[sl], gsems[sl]).wait()
        pltpu.make_async_copy(emb2_hbm.at[cid].at[tsts[sl]], trowss[sl], gsems[sl]).wait()

    def issue_scatter(sl):
        pltpu.async_copy(orowss[sl], acc_sh.at[sidxs[sl]], ssems[sl], add=True)
        pltpu.async_copy(ewbs[sl], den_sh.at[sidxs[sl]], ssems[sl], add=True)

    def wait_scatter(sl):
        pltpu.make_async_copy(orowss[sl], acc_sh.at[sidxs[sl]], ssems[sl]).wait()
        pltpu.make_async_copy(ewbs[sl], den_sh.at[sidxs[sl]], ssems[sl]).wait()

    def prep_indices(sl):
        for c in range(EB // 16):
            s = pl.ds(c * 16, 16)
            h = pbufs[sl][s]
            t = pbufs[sl][pl.ds(EB + c * 16, 16)]
            y = pbufs[sl][pl.ds(2 * EB + c * 16, 16)]
            k = jnp.where(y == 0, 22, y - 1)
            ksts[sl][s] = k
            qi2s[sl][s] = h * 128 + k
            qi2s[sl][pl.ds(EB + c * 16, 16)] = t * 128 + k
            sidxs[sl][s] = h
            tsts[sl][s] = t

    def compute_batch(sl):
        for c in range(EB // 16):
            s = pl.ds(c * 16, 16)
            ew = jnp.exp(qv2s[sl][s] * qv2s[sl][pl.ds(EB + c * 16, 16)])
            ewbs[sl][s] = ew
            kc = ksts[sl][s]
            for l in range(16):
                e = c * 16 + l
                ewv = jnp.full((16,), ew[l], jnp.float32)
                kbase = kc[l] * HALF
                for j in range(HALF // 16):
                    tj = trowss[sl][e, pl.ds(j * 16, 16)]
                    rj = wtab[pl.ds(kbase + j * 16, 16)]
                    orowss[sl][e, pl.ds(j * 16, 16)] = ewv * (tj * rj)

    issue_idx(0, 0)
    issue_idx(1, 1)
    issue_idx(2, 2)

    NB = N_EBATCH

    def loop_body(i, carry):
        for b in range(3):
            bi = 3 * i + b
            sl = b
            cl = (b + 1) % 3

            @pl.when(bi < NB)
            def _():
                wait_idx(bi, sl)
            @pl.when(bi >= 3)
            def _():
                wait_scatter(sl)
            @pl.when(bi < NB)
            def _():
                prep_indices(sl)
                issue_gather(sl)
            @pl.when(bi + 3 < NB)
            def _():
                issue_idx(bi + 3, sl)
            @pl.when(jnp.logical_and(bi >= 2, bi <= NB + 1))
            def _():
                wait_gather(cl)
                compute_batch(cl)
                issue_scatter(cl)
        return carry

    lax.fori_loop(0, (NB + 3) // 3, loop_body, 0)
    plsc.subcore_barrier()

    for jb in range(25):
        b = jb * 16 + sid
        @pl.when(b < ENT_OUT_ROWS // 128)
        def _():
            pltpu.sync_copy(acc_sh.at[pl.ds(b * 128, 128)],
                            num_hbm.at[cid].at[pl.ds(b * 128, 128)])
            pltpu.sync_copy(den_sh.at[pl.ds(b * 128, 128)],
                            den_hbm.at[cid].at[pl.ds(b * 128, 128)])


def _entity_sc(qflat, emb2, edges_packed, w2, zr, zd):
    mesh = plsc.VectorSubcoreMesh(core_axis_name="c", subcore_axis_name="s",
                                  num_cores=NC, num_subcores=NS)
    ib = lambda: pltpu.VMEM((EB,), jnp.int32)
    fb = lambda: pltpu.VMEM((EB,), jnp.float32)
    rb = lambda: pltpu.VMEM((EB, HALF), jnp.float32)
    scratch = [
        pltpu.VMEM_SHARED((ACC_ROWS, HALF), jnp.float32),
        pltpu.VMEM_SHARED((ACC_ROWS,), jnp.float32),
    ]
    scratch += [pltpu.VMEM((3 * EB,), jnp.int32) for _ in range(3)]
    scratch += [pltpu.VMEM((2 * EB,), jnp.int32) for _ in range(3)]
    scratch += [pltpu.VMEM((2 * EB,), jnp.float32) for _ in range(3)]
    scratch += [fb() for _ in range(3)]
    scratch += [rb() for _ in range(6)]
    scratch += [ib() for _ in range(9)]
    scratch += [pltpu.VMEM((R * HALF,), jnp.float32)]
    scratch += [pltpu.SemaphoreType.DMA for _ in range(9)]
    f = pl.kernel(
        _entity_sc_body,
        out_type=(
            jax.ShapeDtypeStruct((NC, ENT_OUT_ROWS, HALF), jnp.float32),
            jax.ShapeDtypeStruct((NC, ENT_OUT_ROWS), jnp.float32),
        ),
        mesh=mesh,
        compiler_params=pltpu.CompilerParams(use_tc_tiling_on_sc=False),
        scratch_types=scratch,
    )
    return f(qflat, emb2, edges_packed, w2, zr, zd)


def _div_body(num_ref, den_ref, o_ref):
    num = num_ref[...]
    den = den_ref[...]
    inv0 = (1.0 / jnp.maximum(den[0], 1e-37))[:, None]
    inv1 = (1.0 / jnp.maximum(den[1], 1e-37))[:, None]
    o_ref[...] = jnp.concatenate([num[0] * inv0, num[1] * inv1], axis=1)


def _entity_div(num, den):
    blk = 1792
    return pl.pallas_call(
        _div_body,
        grid=(ENT_OUT_ROWS // blk,),
        in_specs=[
            pl.BlockSpec((NC, blk, HALF), lambda i: (0, i, 0)),
            pl.BlockSpec((NC, blk), lambda i: (0, i)),
        ],
        out_specs=pl.BlockSpec((blk, D), lambda i: (i, 0)),
        out_shape=jax.ShapeDtypeStruct((ENT_OUT_ROWS, D), jnp.float32),
    )(num, den)


def _user_sc_body(*refs):
    (emb_hbm, nz_hbm, zu_hbm, out_hbm, uacc_sh) = refs[:5]
    r = refs[5:]
    groups = [tuple(r[i * 3:(i + 1) * 3]) for i in range(6)]
    (pbufs, sidxs, vsts, csts, erowss, orowss) = groups
    isems = r[18:21]
    gsems = r[21:24]
    ssems = r[24:27]

    cid = lax.axis_index("c")
    sid = lax.axis_index("s")
    wid = sid * NC + cid

    for j in range(10):
        b = j * 16 + sid
        @pl.when(b < U_BLOCKS)
        def _():
            pltpu.sync_copy(zu_hbm, uacc_sh.at[pl.ds(b * 128, 128)])
    plsc.subcore_barrier()

    def issue_idx(bi, sl):
        pltpu.async_copy(nz_hbm.at[wid].at[pl.ds(bi * 3 * UB, 3 * UB)],
                         pbufs[sl], isems[sl])

    def wait_idx(bi, sl):
        pltpu.make_async_copy(nz_hbm.at[wid].at[pl.ds(bi * 3 * UB, 3 * UB)],
                              pbufs[sl], isems[sl]).wait()

    def issue_gather(sl):
        pltpu.async_copy(emb_hbm.at[csts[sl]], erowss[sl], gsems[sl])

    def wait_gather(sl):
        pltpu.make_async_copy(emb_hbm.at[csts[sl]], erowss[sl], gsems[sl]).wait()

    def issue_scatter(sl):
        pltpu.async_copy(orowss[sl], uacc_sh.at[sidxs[sl]], ssems[sl], add=True)

    def wait_scatter(sl):
        pltpu.make_async_copy(orowss[sl], uacc_sh.at[sidxs[sl]], ssems[sl]).wait()

    def stash(sl):
        for c in range(UB // 16):
            s = pl.ds(c * 16, 16)
            csts[sl][s] = pbufs[sl][s]
            sidxs[sl][s] = pbufs[sl][pl.ds(UB + c * 16, 16)]
            vsts[sl][s] = lax.bitcast_convert_type(
                pbufs[sl][pl.ds(2 * UB + c * 16, 16)], jnp.float32)

    def compute_batch(sl):
        for c in range(UB // 16):
            vc = vsts[sl][pl.ds(c * 16, 16)]
            for l in range(16):
                e = c * 16 + l
                vv = jnp.full((16,), vc[l], jnp.float32)
                for j in range(D // 16):
                    orowss[sl][e, pl.ds(j * 16, 16)] = (
                        vv * erowss[sl][e, pl.ds(j * 16, 16)])

    issue_idx(0, 0)
    issue_idx(1, 1)
    issue_idx(2, 2)

    NB = N_UBATCH

    def loop_body(i, carry):
        for b in range(3):
            bi = 3 * i + b
            sl = b
            cl = (b + 1) % 3

            @pl.when(bi < NB)
            def _():
                wait_idx(bi, sl)
            @pl.when(bi >= 3)
            def _():
                wait_scatter(sl)
            @pl.when(bi < NB)
            def _():
                stash(sl)
                issue_gather(sl)
            @pl.when(bi + 3 < NB)
            def _():
                issue_idx(bi + 3, sl)
            @pl.when(jnp.logical_and(bi >= 2, bi <= NB + 1))
            def _():
                wait_gather(cl)
                compute_batch(cl)
                issue_scatter(cl)
        return carry

    lax.fori_loop(0, (NB + 3) // 3, loop_body, 0)
    plsc.subcore_barrier()

    for j in range(10):
        b = j * 16 + sid
        @pl.when(b < U_BLOCKS)
        def _():
            pltpu.sync_copy(uacc_sh.at[pl.ds(b * 128, 128)],
                            out_hbm.at[cid].at[pl.ds(b * 128, 128)])


def _user_sc(emb, nz_packed, zu):
    mesh = plsc.VectorSubcoreMesh(core_axis_name="c", subcore_axis_name="s",
                                  num_cores=NC, num_subcores=NS)
    ib = lambda: pltpu.VMEM((UB,), jnp.int32)
    fb = lambda: pltpu.VMEM((UB,), jnp.float32)
    db = lambda: pltpu.VMEM((UB, D), jnp.float32)
    scratch = [pltpu.VMEM_SHARED((U_ROWS, D), jnp.float32)]
    scratch += [pltpu.VMEM((3 * UB,), jnp.int32) for _ in range(3)]
    scratch += [ib() for _ in range(3)]
    scratch += [fb() for _ in range(3)]
    scratch += [ib() for _ in range(3)]
    scratch += [db() for _ in range(6)]
    scratch += [pltpu.SemaphoreType.DMA for _ in range(9)]
    f = pl.kernel(
        _user_sc_body,
        out_type=jax.ShapeDtypeStruct((NC, U_ROWS, D), jnp.float32),
        mesh=mesh,
        compiler_params=pltpu.CompilerParams(use_tc_tiling_on_sc=False),
        scratch_types=scratch,
    )
    return f(emb, nz_packed, zu)


def _ep_body(ue_ref, w_ref, p_ref, o_ref):
    ue = ue_ref[...]
    w = w_ref[...]
    logits = lax.dot_general(ue, w, (((1,), (1,)), ((), ())),
                             preferred_element_type=jnp.float32)
    col = lax.broadcasted_iota(jnp.int32, logits.shape, 1)
    logits = jnp.where(col < R, logits, -1e30)
    m = jnp.max(logits, axis=-1, keepdims=True)
    ex = jnp.exp(logits - m)
    score = ex / jnp.sum(ex, axis=-1, keepdims=True)
    mult = jnp.dot(score, w, preferred_element_type=jnp.float32)
    p = p_ref[0] + p_ref[1]
    o_ref[...] = p * (1.0 + mult)


def _epilogue(user_emb, w24, partials):
    blk = 1000
    return pl.pallas_call(
        _ep_body,
        grid=(NU // blk,),
        in_specs=[
            pl.BlockSpec((blk, D), lambda i: (i, 0)),
            pl.BlockSpec((24, D), lambda i: (0, 0)),
            pl.BlockSpec((NC, blk, D), lambda i: (0, i, 0)),
        ],
        out_specs=pl.BlockSpec((blk, D), lambda i: (i, 0)),
        out_shape=jax.ShapeDtypeStruct((NU, D), jnp.float32),
    )(user_emb, w24, partials)


def kernel(entity_emb, user_emb, edge_index, edge_type, interact_rows,
           interact_cols, interact_values, weight):
    i32 = jnp.int32
    head = edge_index[0].astype(i32)
    tail = edge_index[1].astype(i32)
    etype = edge_type.astype(i32)

    emb_pad = jnp.pad(entity_emb, ((0, QROWS - NE), (0, 0)))
    wpad = jnp.pad(weight, ((0, 128 - R), (0, 0)))
    q = _compute_q(emb_pad, wpad)
    qflat = q.reshape(-1)

    npad = E_PAD - E_REAL
    head_p = jnp.concatenate([head, jnp.full((npad,), NE, i32)])
    tail_p = jnp.concatenate([tail, jnp.zeros((npad,), i32)])
    y_p = jnp.concatenate([etype, jnp.ones((npad,), i32)])
    edges_packed = jnp.stack(
        [head_p.reshape(NS, N_EBATCH, EB),
         tail_p.reshape(NS, N_EBATCH, EB),
         y_p.reshape(NS, N_EBATCH, EB)], axis=2).reshape(NS, N_EBATCH * 3 * EB)

    emb2 = jnp.stack([entity_emb[:, :HALF], entity_emb[:, HALF:]])
    w2 = jnp.stack([weight[:, :HALF].reshape(-1), weight[:, HALF:].reshape(-1)])

    zr = jnp.zeros((128, HALF), jnp.float32)
    zd = jnp.zeros((128,), jnp.float32)
    num, den = _entity_sc(qflat, emb2, edges_packed, w2, zr, zd)
    entity_agg = _entity_div(num, den)[:NE]

    upad = NNZ_PAD - NNZ_REAL
    cols_p = jnp.concatenate([interact_cols.astype(i32), jnp.zeros((upad,), i32)])
    rows_p = jnp.concatenate([interact_rows.astype(i32), jnp.zeros((upad,), i32)])
    vals_p = jnp.concatenate([interact_values, jnp.zeros((upad,), jnp.float32)])
    nz_packed = jnp.stack(
        [cols_p.reshape(NW, N_UBATCH, UB),
         rows_p.reshape(NW, N_UBATCH, UB),
         lax.bitcast_convert_type(vals_p, i32).reshape(NW, N_UBATCH, UB)],
        axis=2).reshape(NW, N_UBATCH * 3 * UB)
    zu = jnp.zeros((128, D), jnp.float32)
    partials = _user_sc(entity_emb, nz_packed, zu)

    w24 = jnp.pad(weight, ((0, 1), (0, 0)))
    user_agg = _epilogue(user_emb, w24, partials)
    return (entity_agg, user_agg)


# speedup vs baseline: 11.0148x; 1.1224x over previous
import functools

import jax
import jax.numpy as jnp
from jax import lax
from jax.experimental import pallas as pl
from jax.experimental.pallas import tpu as pltpu
from jax.experimental.pallas import tpu_sc as plsc

NE = 50000
NU = 20000
D = 64
R = 23
NC, NS = 2, 16
NW = NC * NS

E_REAL = 800000
E_PAD = 801024
EDGES_PER_SC_WORKER = E_PAD // NS
EB = 112
N_EBATCH = EDGES_PER_SC_WORKER // EB
HALF = D // 2
ENT_OUT_ROWS = 50176
ACC_ROWS = 50176
QROWS = 50176

NNZ_REAL = 500000
NNZ_PAD = 505344
NNZ_PER_WORKER = NNZ_PAD // NW
UB = 112
N_UBATCH = NNZ_PER_WORKER // UB
U_BLOCKS = 157
U_ROWS = U_BLOCKS * 128


def _splat_lane(v, l):
    idx = jnp.full((16, 1), l, jnp.int32)
    dnums = lax.GatherDimensionNumbers(
        offset_dims=(), collapsed_slice_dims=(0,), start_index_map=(0,))
    return lax.gather(v, idx, dnums, (1,),
                      mode=lax.GatherScatterMode.PROMISE_IN_BOUNDS)


def _q_body(x_ref, w_ref, o_ref):
    x = x_ref[...]
    w = w_ref[...]
    o_ref[...] = lax.dot_general(x * x, w * w, (((1,), (1,)), ((), ())),
                                 preferred_element_type=jnp.float32)


def _compute_q(emb_pad, wpad):
    blk = 1792
    return pl.pallas_call(
        _q_body,
        grid=(QROWS // blk,),
        in_specs=[
            pl.BlockSpec((blk, D), lambda i: (i, 0)),
            pl.BlockSpec((128, D), lambda i: (0, 0)),
        ],
        out_specs=pl.BlockSpec((blk, 128), lambda i: (i, 0)),
        out_shape=jax.ShapeDtypeStruct((QROWS, 128), jnp.float32),
    )(emb_pad, wpad)


def _entity_sc_body(*refs):
    (qflat_hbm, emb2_hbm, edges_hbm, w2_hbm, zr_hbm, zd_hbm,
     num_hbm, den_hbm, acc_sh, den_sh) = refs[:10]
    r = refs[10:]
    groups = [tuple(r[i * 3:(i + 1) * 3]) for i in range(8)]
    (pbufs, qi2s, qv2s, ewbs, trowss, orowss, sidxs, ksts) = groups
    tsts = tuple(r[24:27])
    wtab = r[27]
    isems = r[28:31]
    gsems = r[31:34]
    ssems = r[34:37]

    cid = lax.axis_index("c")
    sid = lax.axis_index("s")

    pltpu.sync_copy(w2_hbm.at[cid], wtab)

    for j in range(25):
        b = j * 16 + sid
        @pl.when(b < ACC_ROWS // 128)
        def _():
            pltpu.sync_copy(zr_hbm, acc_sh.at[pl.ds(b * 128, 128)])
            pltpu.sync_copy(zd_hbm, den_sh.at[pl.ds(b * 128, 128)])
    plsc.subcore_barrier()

    def issue_idx(bi, sl):
        pltpu.async_copy(edges_hbm.at[sid].at[pl.ds(bi * 3 * EB, 3 * EB)],
                         pbufs[sl], isems[sl])

    def wait_idx(bi, sl):
        pltpu.make_async_copy(edges_hbm.at[sid].at[pl.ds(bi * 3 * EB, 3 * EB)],
                              pbufs[sl], isems[sl]).wait()

    def issue_gather(sl):
        pltpu.async_copy(qflat_hbm.at[qi2s[sl]], qv2s[sl], gsems[sl])
        pltpu.async_copy(emb2_hbm.at[cid].at[tsts[sl]], trowss[sl], gsems[sl])

    def wait_gather(sl):
        pltpu.make_async_copy(qflat_hbm.at[qi2s[sl]], qv2s[sl], gsems[sl]).wait()
        pltpu.make_async_copy(emb2_hbm.at[cid].at[tsts[sl]], trowss[sl], gsems[sl]).wait()

    def issue_scatter(sl):
        pltpu.async_copy(orowss[sl], acc_sh.at[sidxs[sl]], ssems[sl], add=True)
        pltpu.async_copy(ewbs[sl], den_sh.at[sidxs[sl]], ssems[sl], add=True)

    def wait_scatter(sl):
        pltpu.make_async_copy(orowss[sl], acc_sh.at[sidxs[sl]], ssems[sl]).wait()
        pltpu.make_async_copy(ewbs[sl], den_sh.at[sidxs[sl]], ssems[sl]).wait()

    def prep_indices(sl):
        for c in range(EB // 16):
            s = pl.ds(c * 16, 16)
            h = pbufs[sl][s]
            t = pbufs[sl][pl.ds(EB + c * 16, 16)]
            y = pbufs[sl][pl.ds(2 * EB + c * 16, 16)]
            k = jnp.where(y == 0, 22, y - 1)
            ksts[sl][s] = k
            qi2s[sl][s] = h * 128 + k
            qi2s[sl][pl.ds(EB + c * 16, 16)] = t * 128 + k
            sidxs[sl][s] = h
            tsts[sl][s] = t

    def compute_batch(sl):
        iota0 = jnp.arange(16, dtype=jnp.int32)
        iota1 = iota0 + 16
        for c in range(EB // 16):
            s = pl.ds(c * 16, 16)
            ew = jnp.exp(qv2s[sl][s] * qv2s[sl][pl.ds(EB + c * 16, 16)])
            ewbs[sl][s] = ew
            kb = ksts[sl][s] * HALF
            for l in range(16):
                e = c * 16 + l
                ewv = _splat_lane(ew, l)
                kv = _splat_lane(kb, l)
                r0 = plsc.load_gather(wtab, [kv + iota0])
                r1 = plsc.load_gather(wtab, [kv + iota1])
                t0 = trowss[sl][e, pl.ds(0, 16)]
                t1 = trowss[sl][e, pl.ds(16, 16)]
                orowss[sl][e, pl.ds(0, 16)] = ewv * (t0 * r0)
                orowss[sl][e, pl.ds(16, 16)] = ewv * (t1 * r1)

    issue_idx(0, 0)
    issue_idx(1, 1)
    issue_idx(2, 2)

    NB = N_EBATCH

    def loop_body(i, carry):
        for b in range(3):
            bi = 3 * i + b
            sl = b
            cl = (b + 1) % 3

            @pl.when(bi < NB)
            def _():
                wait_idx(bi, sl)
            @pl.when(bi >= 3)
            def _():
                wait_scatter(sl)
            @pl.when(bi < NB)
            def _():
                prep_indices(sl)
                issue_gather(sl)
            @pl.when(bi + 3 < NB)
            def _():
                issue_idx(bi + 3, sl)
            @pl.when(jnp.logical_and(bi >= 2, bi <= NB + 1))
            def _():
                wait_gather(cl)
                compute_batch(cl)
                issue_scatter(cl)
        return carry

    lax.fori_loop(0, (NB + 3) // 3, loop_body, 0)
    plsc.subcore_barrier()

    for jb in range(25):
        b = jb * 16 + sid
        @pl.when(b < ENT_OUT_ROWS // 128)
        def _():
            pltpu.sync_copy(acc_sh.at[pl.ds(b * 128, 128)],
                            num_hbm.at[cid].at[pl.ds(b * 128, 128)])
            pltpu.sync_copy(den_sh.at[pl.ds(b * 128, 128)],
                            den_hbm.at[cid].at[pl.ds(b * 128, 128)])


def _entity_sc(qflat, emb2, edges_packed, w2, zr, zd):
    mesh = plsc.VectorSubcoreMesh(core_axis_name="c", subcore_axis_name="s",
                                  num_cores=NC, num_subcores=NS)
    ib = lambda: pltpu.VMEM((EB,), jnp.int32)
    fb = lambda: pltpu.VMEM((EB,), jnp.float32)
    rb = lambda: pltpu.VMEM((EB, HALF), jnp.float32)
    scratch = [
        pltpu.VMEM_SHARED((ACC_ROWS, HALF), jnp.float32),
        pltpu.VMEM_SHARED((ACC_ROWS,), jnp.float32),
    ]
    scratch += [pltpu.VMEM((3 * EB,), jnp.int32) for _ in range(3)]
    scratch += [pltpu.VMEM((2 * EB,), jnp.int32) for _ in range(3)]
    scratch += [pltpu.VMEM((2 * EB,), jnp.float32) for _ in range(3)]
    scratch += [fb() for _ in range(3)]
    scratch += [rb() for _ in range(6)]
    scratch += [ib() for _ in range(9)]
    scratch += [pltpu.VMEM((R * HALF,), jnp.float32)]
    scratch += [pltpu.SemaphoreType.DMA for _ in range(9)]
    f = pl.kernel(
        _entity_sc_body,
        out_type=(
            jax.ShapeDtypeStruct((NC, ENT_OUT_ROWS, HALF), jnp.float32),
            jax.ShapeDtypeStruct((NC, ENT_OUT_ROWS), jnp.float32),
        ),
        mesh=mesh,
        compiler_params=pltpu.CompilerParams(use_tc_tiling_on_sc=False, needs_layout_passes=False),
        scratch_types=scratch,
    )
    return f(qflat, emb2, edges_packed, w2, zr, zd)


def _div_body(num_ref, den_ref, o_ref):
    num = num_ref[...]
    den = den_ref[...]
    inv0 = (1.0 / jnp.maximum(den[0], 1e-37))[:, None]
    inv1 = (1.0 / jnp.maximum(den[1], 1e-37))[:, None]
    o_ref[...] = jnp.concatenate([num[0] * inv0, num[1] * inv1], axis=1)


def _entity_div(num, den):
    blk = 1792
    return pl.pallas_call(
        _div_body,
        grid=(ENT_OUT_ROWS // blk,),
        in_specs=[
            pl.BlockSpec((NC, blk, HALF), lambda i: (0, i, 0)),
            pl.BlockSpec((NC, blk), lambda i: (0, i)),
        ],
        out_specs=pl.BlockSpec((blk, D), lambda i: (i, 0)),
        out_shape=jax.ShapeDtypeStruct((ENT_OUT_ROWS, D), jnp.float32),
    )(num, den)


def _user_sc_body(*refs):
    (emb_hbm, nz_hbm, zu_hbm, out_hbm, uacc_sh) = refs[:5]
    r = refs[5:]
    groups = [tuple(r[i * 3:(i + 1) * 3]) for i in range(6)]
    (pbufs, sidxs, vsts, csts, erowss, orowss) = groups
    isems = r[18:21]
    gsems = r[21:24]
    ssems = r[24:27]

    cid = lax.axis_index("c")
    sid = lax.axis_index("s")
    wid = sid * NC + cid

    for j in range(10):
        b = j * 16 + sid
        @pl.when(b < U_BLOCKS)
        def _():
            pltpu.sync_copy(zu_hbm, uacc_sh.at[pl.ds(b * 128, 128)])
    plsc.subcore_barrier()

    def issue_idx(bi, sl):
        pltpu.async_copy(nz_hbm.at[wid].at[pl.ds(bi * 3 * UB, 3 * UB)],
                         pbufs[sl], isems[sl])

    def wait_idx(bi, sl):
        pltpu.make_async_copy(nz_hbm.at[wid].at[pl.ds(bi * 3 * UB, 3 * UB)],
                              pbufs[sl], isems[sl]).wait()

    def issue_gather(sl):
        pltpu.async_copy(emb_hbm.at[csts[sl]], erowss[sl], gsems[sl])

    def wait_gather(sl):
        pltpu.make_async_copy(emb_hbm.at[csts[sl]], erowss[sl], gsems[sl]).wait()

    def issue_scatter(sl):
        pltpu.async_copy(orowss[sl], uacc_sh.at[sidxs[sl]], ssems[sl], add=True)

    def wait_scatter(sl):
        pltpu.make_async_copy(orowss[sl], uacc_sh.at[sidxs[sl]], ssems[sl]).wait()

    def stash(sl):
        for c in range(UB // 16):
            s = pl.ds(c * 16, 16)
            csts[sl][s] = pbufs[sl][s]
            sidxs[sl][s] = pbufs[sl][pl.ds(UB + c * 16, 16)]
            vsts[sl][s] = lax.bitcast_convert_type(
                pbufs[sl][pl.ds(2 * UB + c * 16, 16)], jnp.float32)

    def compute_batch(sl):
        for c in range(UB // 16):
            vc = vsts[sl][pl.ds(c * 16, 16)]
            for l in range(16):
                e = c * 16 + l
                vv = _splat_lane(vc, l)
                for j in range(D // 16):
                    orowss[sl][e, pl.ds(j * 16, 16)] = (
                        vv * erowss[sl][e, pl.ds(j * 16, 16)])

    issue_idx(0, 0)
    issue_idx(1, 1)
    issue_idx(2, 2)

    NB = N_UBATCH

    def loop_body(i, carry):
        for b in range(3):
            bi = 3 * i + b
            sl = b
            cl = (b + 1) % 3

            @pl.when(bi < NB)
            def _():
                wait_idx(bi, sl)
            @pl.when(bi >= 3)
            def _():
                wait_scatter(sl)
            @pl.when(bi < NB)
            def _():
                stash(sl)
                issue_gather(sl)
            @pl.when(bi + 3 < NB)
            def _():
                issue_idx(bi + 3, sl)
            @pl.when(jnp.logical_and(bi >= 2, bi <= NB + 1))
            def _():
                wait_gather(cl)
                compute_batch(cl)
                issue_scatter(cl)
        return carry

    lax.fori_loop(0, (NB + 3) // 3, loop_body, 0)
    plsc.subcore_barrier()

    for j in range(10):
        b = j * 16 + sid
        @pl.when(b < U_BLOCKS)
        def _():
            pltpu.sync_copy(uacc_sh.at[pl.ds(b * 128, 128)],
                            out_hbm.at[cid].at[pl.ds(b * 128, 128)])


def _user_sc(emb, nz_packed, zu):
    mesh = plsc.VectorSubcoreMesh(core_axis_name="c", subcore_axis_name="s",
                                  num_cores=NC, num_subcores=NS)
    ib = lambda: pltpu.VMEM((UB,), jnp.int32)
    fb = lambda: pltpu.VMEM((UB,), jnp.float32)
    db = lambda: pltpu.VMEM((UB, D), jnp.float32)
    scratch = [pltpu.VMEM_SHARED((U_ROWS, D), jnp.float32)]
    scratch += [pltpu.VMEM((3 * UB,), jnp.int32) for _ in range(3)]
    scratch += [ib() for _ in range(3)]
    scratch += [fb() for _ in range(3)]
    scratch += [ib() for _ in range(3)]
    scratch += [db() for _ in range(6)]
    scratch += [pltpu.SemaphoreType.DMA for _ in range(9)]
    f = pl.kernel(
        _user_sc_body,
        out_type=jax.ShapeDtypeStruct((NC, U_ROWS, D), jnp.float32),
        mesh=mesh,
        compiler_params=pltpu.CompilerParams(use_tc_tiling_on_sc=False, needs_layout_passes=False),
        scratch_types=scratch,
    )
    return f(emb, nz_packed, zu)


def _ep_body(ue_ref, w_ref, p_ref, o_ref):
    ue = ue_ref[...]
    w = w_ref[...]
    logits = lax.dot_general(ue, w, (((1,), (1,)), ((), ())),
                             preferred_element_type=jnp.float32)
    col = lax.broadcasted_iota(jnp.int32, logits.shape, 1)
    logits = jnp.where(col < R, logits, -1e30)
    m = jnp.max(logits, axis=-1, keepdims=True)
    ex = jnp.exp(logits - m)
    score = ex / jnp.sum(ex, axis=-1, keepdims=True)
    mult = jnp.dot(score, w, preferred_element_type=jnp.float32)
    p = p_ref[0] + p_ref[1]
    o_ref[...] = p * (1.0 + mult)


def _epilogue(user_emb, w24, partials):
    blk = 1000
    return pl.pallas_call(
        _ep_body,
        grid=(NU // blk,),
        in_specs=[
            pl.BlockSpec((blk, D), lambda i: (i, 0)),
            pl.BlockSpec((24, D), lambda i: (0, 0)),
            pl.BlockSpec((NC, blk, D), lambda i: (0, i, 0)),
        ],
        out_specs=pl.BlockSpec((blk, D), lambda i: (i, 0)),
        out_shape=jax.ShapeDtypeStruct((NU, D), jnp.float32),
    )(user_emb, w24, partials)


def kernel(entity_emb, user_emb, edge_index, edge_type, interact_rows,
           interact_cols, interact_values, weight):
    i32 = jnp.int32
    head = edge_index[0].astype(i32)
    tail = edge_index[1].astype(i32)
    etype = edge_type.astype(i32)

    emb_pad = jnp.pad(entity_emb, ((0, QROWS - NE), (0, 0)))
    wpad = jnp.pad(weight, ((0, 128 - R), (0, 0)))
    q = _compute_q(emb_pad, wpad)
    qflat = q.reshape(-1)

    npad = E_PAD - E_REAL
    head_p = jnp.concatenate([head, jnp.full((npad,), NE, i32)])
    tail_p = jnp.concatenate([tail, jnp.zeros((npad,), i32)])
    y_p = jnp.concatenate([etype, jnp.ones((npad,), i32)])
    edges_packed = jnp.stack(
        [head_p.reshape(NS, N_EBATCH, EB),
         tail_p.reshape(NS, N_EBATCH, EB),
         y_p.reshape(NS, N_EBATCH, EB)], axis=2).reshape(NS, N_EBATCH * 3 * EB)

    emb2 = jnp.stack([entity_emb[:, :HALF], entity_emb[:, HALF:]])
    w2 = jnp.stack([weight[:, :HALF].reshape(-1), weight[:, HALF:].reshape(-1)])

    zr = jnp.zeros((128, HALF), jnp.float32)
    zd = jnp.zeros((128,), jnp.float32)
    num, den = _entity_sc(qflat, emb2, edges_packed, w2, zr, zd)
    entity_agg = _entity_div(num, den)[:NE]

    upad = NNZ_PAD - NNZ_REAL
    cols_p = jnp.concatenate([interact_cols.astype(i32), jnp.zeros((upad,), i32)])
    rows_p = jnp.concatenate([interact_rows.astype(i32), jnp.zeros((upad,), i32)])
    vals_p = jnp.concatenate([interact_values, jnp.zeros((upad,), jnp.float32)])
    nz_packed = jnp.stack(
        [cols_p.reshape(NW, N_UBATCH, UB),
         rows_p.reshape(NW, N_UBATCH, UB),
         lax.bitcast_convert_type(vals_p, i32).reshape(NW, N_UBATCH, UB)],
        axis=2).reshape(NW, N_UBATCH * 3 * UB)
    zu = jnp.zeros((128, D), jnp.float32)
    partials = _user_sc(entity_emb, nz_packed, zu)

    w24 = jnp.pad(weight, ((0, 1), (0, 0)))
    user_agg = _epilogue(user_emb, w24, partials)
    return (entity_agg, user_agg)


# speedup vs baseline: 11.1114x; 1.0088x over previous
import jax
import jax.numpy as jnp
from jax import lax
from jax.experimental import pallas as pl
from jax.experimental.pallas import tpu as pltpu
from jax.experimental.pallas import tpu_sc as plsc

NE = 50000
NU = 20000
D = 64
R = 23
NC, NS = 2, 16
NW = NC * NS

E_REAL = 800000
E_PAD = 801024
EDGES_PER_SC_WORKER = E_PAD // NS
EB = 112
N_EBATCH = EDGES_PER_SC_WORKER // EB
HALF = D // 2
ENT_OUT_ROWS = 50176
ACC_ROWS = 50176
QROWS = 50176

NNZ_REAL = 500000
NNZ_PAD = 505344
NNZ_PER_WORKER = NNZ_PAD // NW
UB = 112
N_UBATCH = NNZ_PER_WORKER // UB
U_BLOCKS = 157
U_ROWS = U_BLOCKS * 128


def _splat_lane(v, l):
    idx = jnp.full((16, 1), l, jnp.int32)
    dnums = lax.GatherDimensionNumbers(
        offset_dims=(), collapsed_slice_dims=(0,), start_index_map=(0,))
    return lax.gather(v, idx, dnums, (1,),
                      mode=lax.GatherScatterMode.PROMISE_IN_BOUNDS)


def _q_body(x_ref, w_ref, o_ref):
    x = x_ref[...]
    w = w_ref[...]
    o_ref[...] = lax.dot_general(x * x, w * w, (((1,), (1,)), ((), ())),
                                 preferred_element_type=jnp.float32)


def _compute_q(emb_pad, wpad):
    blk = 1792
    return pl.pallas_call(
        _q_body,
        grid=(QROWS // blk,),
        in_specs=[
            pl.BlockSpec((blk, D), lambda i: (i, 0)),
            pl.BlockSpec((128, D), lambda i: (0, 0)),
        ],
        out_specs=pl.BlockSpec((blk, 128), lambda i: (i, 0)),
        out_shape=jax.ShapeDtypeStruct((QROWS, 128), jnp.float32),
    )(emb_pad, wpad)


def _entity_sc_body(*refs):
    (qflat_hbm, emb2_hbm, edges_hbm, w2_hbm, zr_hbm, zd_hbm,
     num_hbm, den_hbm, acc_sh, den_sh) = refs[:10]
    r = refs[10:]
    groups = [tuple(r[i * 3:(i + 1) * 3]) for i in range(8)]
    (pbufs, qi2s, qv2s, ewbs, trowss, orowss, sidxs, ksts) = groups
    tsts = tuple(r[24:27])
    wtab = r[27]
    isems = r[28:31]
    gsems = r[31:34]
    ssems = r[34:37]

    cid = lax.axis_index("c")
    sid = lax.axis_index("s")

    pltpu.sync_copy(w2_hbm.at[cid], wtab)

    for j in range(25):
        b = j * 16 + sid
        @pl.when(b < ACC_ROWS // 128)
        def _():
            pltpu.sync_copy(zr_hbm, acc_sh.at[pl.ds(b * 128, 128)])
            pltpu.sync_copy(zd_hbm, den_sh.at[pl.ds(b * 128, 128)])
    plsc.subcore_barrier()

    def issue_idx(bi, sl):
        pltpu.async_copy(edges_hbm.at[sid].at[pl.ds(bi * 3 * EB, 3 * EB)],
                         pbufs[sl], isems[sl])

    def wait_idx(bi, sl):
        pltpu.make_async_copy(edges_hbm.at[sid].at[pl.ds(bi * 3 * EB, 3 * EB)],
                              pbufs[sl], isems[sl]).wait()

    def issue_gather(sl):
        pltpu.async_copy(qflat_hbm.at[qi2s[sl]], qv2s[sl], gsems[sl])
        pltpu.async_copy(emb2_hbm.at[cid].at[tsts[sl]], trowss[sl], gsems[sl])

    def wait_gather(sl):
        pltpu.make_async_copy(qflat_hbm.at[qi2s[sl]], qv2s[sl], gsems[sl]).wait()
        pltpu.make_async_copy(emb2_hbm.at[cid].at[tsts[sl]], trowss[sl], gsems[sl]).wait()

    def issue_scatter(sl):
        pltpu.async_copy(orowss[sl], acc_sh.at[sidxs[sl]], ssems[sl], add=True)
        pltpu.async_copy(ewbs[sl], den_sh.at[sidxs[sl]], ssems[sl], add=True)

    def wait_scatter(sl):
        pltpu.make_async_copy(orowss[sl], acc_sh.at[sidxs[sl]], ssems[sl]).wait()
        pltpu.make_async_copy(ewbs[sl], den_sh.at[sidxs[sl]], ssems[sl]).wait()

    def prep_indices(sl):
        for c in range(EB // 16):
            s = pl.ds(c * 16, 16)
            h = pbufs[sl][s]
            t = pbufs[sl][pl.ds(EB + c * 16, 16)]
            y = pbufs[sl][pl.ds(2 * EB + c * 16, 16)]
            k = jnp.where(y == 0, 22, y - 1)
            ksts[sl][s] = k
            qi2s[sl][s] = h * 128 + k
            qi2s[sl][pl.ds(EB + c * 16, 16)] = t * 128 + k
            sidxs[sl][s] = h
            tsts[sl][s] = t

    def compute_batch(sl):
        iota0 = jnp.arange(16, dtype=jnp.int32)
        iota1 = iota0 + 16
        for c in range(EB // 16):
            s = pl.ds(c * 16, 16)
            ew = jnp.exp(qv2s[sl][s] * qv2s[sl][pl.ds(EB + c * 16, 16)])
            ewbs[sl][s] = ew
            kb = ksts[sl][s] * HALF
            for l in range(16):
                e = c * 16 + l
                ewv = _splat_lane(ew, l)
                kv = _splat_lane(kb, l)
                r0 = plsc.load_gather(wtab, [kv + iota0])
                r1 = plsc.load_gather(wtab, [kv + iota1])
                t0 = trowss[sl][e, pl.ds(0, 16)]
                t1 = trowss[sl][e, pl.ds(16, 16)]
                orowss[sl][e, pl.ds(0, 16)] = ewv * (t0 * r0)
                orowss[sl][e, pl.ds(16, 16)] = ewv * (t1 * r1)

    issue_idx(0, 0)
    issue_idx(1, 1)
    issue_idx(2, 2)

    NB = N_EBATCH

    def loop_body(i, carry):
        for b in range(3):
            bi = 3 * i + b
            sl = b
            cl = (b + 1) % 3

            @pl.when(bi < NB)
            def _():
                wait_idx(bi, sl)
            @pl.when(bi >= 3)
            def _():
                wait_scatter(sl)
            @pl.when(bi < NB)
            def _():
                prep_indices(sl)
                issue_gather(sl)
            @pl.when(bi + 3 < NB)
            def _():
                issue_idx(bi + 3, sl)
            @pl.when(jnp.logical_and(bi >= 2, bi <= NB + 1))
            def _():
                wait_gather(cl)
                compute_batch(cl)
                issue_scatter(cl)
        return carry

    lax.fori_loop(0, (NB + 3) // 3, loop_body, 0)
    plsc.subcore_barrier()

    for jb in range(25):
        b = jb * 16 + sid
        @pl.when(b < ENT_OUT_ROWS // 128)
        def _():
            pltpu.sync_copy(acc_sh.at[pl.ds(b * 128, 128)],
                            num_hbm.at[cid].at[pl.ds(b * 128, 128)])
            pltpu.sync_copy(den_sh.at[pl.ds(b * 128, 128)],
                            den_hbm.at[cid].at[pl.ds(b * 128, 128)])


def _entity_sc(qflat, emb2, edges_packed, w2, zr, zd):
    mesh = plsc.VectorSubcoreMesh(core_axis_name="c", subcore_axis_name="s",
                                  num_cores=NC, num_subcores=NS)
    ib = lambda: pltpu.VMEM((EB,), jnp.int32)
    fb = lambda: pltpu.VMEM((EB,), jnp.float32)
    rb = lambda: pltpu.VMEM((EB, HALF), jnp.float32)
    scratch = [
        pltpu.VMEM_SHARED((ACC_ROWS, HALF), jnp.float32),
        pltpu.VMEM_SHARED((ACC_ROWS,), jnp.float32),
    ]
    scratch += [pltpu.VMEM((3 * EB,), jnp.int32) for _ in range(3)]
    scratch += [pltpu.VMEM((2 * EB,), jnp.int32) for _ in range(3)]
    scratch += [pltpu.VMEM((2 * EB,), jnp.float32) for _ in range(3)]
    scratch += [fb() for _ in range(3)]
    scratch += [rb() for _ in range(6)]
    scratch += [ib() for _ in range(9)]
    scratch += [pltpu.VMEM((R * HALF,), jnp.float32)]
    scratch += [pltpu.SemaphoreType.DMA for _ in range(9)]
    f = pl.kernel(
        _entity_sc_body,
        out_type=(
            jax.ShapeDtypeStruct((NC, ENT_OUT_ROWS, HALF), jnp.float32),
            jax.ShapeDtypeStruct((NC, ENT_OUT_ROWS), jnp.float32),
        ),
        mesh=mesh,
        compiler_params=pltpu.CompilerParams(use_tc_tiling_on_sc=False, needs_layout_passes=False),
        scratch_types=scratch,
    )
    return f(qflat, emb2, edges_packed, w2, zr, zd)


def _div_body(num_ref, den_ref, o_ref):
    num = num_ref[...]
    den = den_ref[...]
    inv0 = (1.0 / jnp.maximum(den[0], 1e-37))[:, None]
    inv1 = (1.0 / jnp.maximum(den[1], 1e-37))[:, None]
    o_ref[...] = jnp.concatenate([num[0] * inv0, num[1] * inv1], axis=1)


def _entity_div(num, den):
    blk = 1792
    return pl.pallas_call(
        _div_body,
        grid=(ENT_OUT_ROWS // blk,),
        in_specs=[
            pl.BlockSpec((NC, blk, HALF), lambda i: (0, i, 0)),
            pl.BlockSpec((NC, blk), lambda i: (0, i)),
        ],
        out_specs=pl.BlockSpec((blk, D), lambda i: (i, 0)),
        out_shape=jax.ShapeDtypeStruct((ENT_OUT_ROWS, D), jnp.float32),
    )(num, den)


def _user_sc_body(*refs):
    (emb_hbm, nz_hbm, zu_hbm, out_hbm, uacc_sh) = refs[:5]
    r = refs[5:]
    groups = [tuple(r[i * 3:(i + 1) * 3]) for i in range(6)]
    (pbufs, sidxs, vsts, csts, erowss, orowss) = groups
    isems = r[18:21]
    gsems = r[21:24]
    ssems = r[24:27]

    cid = lax.axis_index("c")
    sid = lax.axis_index("s")
    wid = sid * NC + cid

    for j in range(10):
        b = j * 16 + sid
        @pl.when(b < U_BLOCKS)
        def _():
            pltpu.sync_copy(zu_hbm, uacc_sh.at[pl.ds(b * 128, 128)])
    plsc.subcore_barrier()

    def issue_idx(bi, sl):
        pltpu.async_copy(nz_hbm.at[wid].at[pl.ds(bi * 3 * UB, 3 * UB)],
                         pbufs[sl], isems[sl])

    def wait_idx(bi, sl):
        pltpu.make_async_copy(nz_hbm.at[wid].at[pl.ds(bi * 3 * UB, 3 * UB)],
                              pbufs[sl], isems[sl]).wait()

    def issue_gather(sl):
        pltpu.async_copy(emb_hbm.at[csts[sl]], erowss[sl], gsems[sl])

    def wait_gather(sl):
        pltpu.make_async_copy(emb_hbm.at[csts[sl]], erowss[sl], gsems[sl]).wait()

    def issue_scatter(sl):
        pltpu.async_copy(orowss[sl], uacc_sh.at[sidxs[sl]], ssems[sl], add=True)

    def wait_scatter(sl):
        pltpu.make_async_copy(orowss[sl], uacc_sh.at[sidxs[sl]], ssems[sl]).wait()

    def stash(sl):
        for c in range(UB // 16):
            s = pl.ds(c * 16, 16)
            csts[sl][s] = pbufs[sl][s]
            sidxs[sl][s] = pbufs[sl][pl.ds(UB + c * 16, 16)]
            vsts[sl][s] = lax.bitcast_convert_type(
                pbufs[sl][pl.ds(2 * UB + c * 16, 16)], jnp.float32)

    def compute_batch(sl):
        for c in range(UB // 16):
            vc = vsts[sl][pl.ds(c * 16, 16)]
            for l in range(16):
                e = c * 16 + l
                vv = _splat_lane(vc, l)
                for j in range(D // 16):
                    orowss[sl][e, pl.ds(j * 16, 16)] = (
                        vv * erowss[sl][e, pl.ds(j * 16, 16)])

    issue_idx(0, 0)
    issue_idx(1, 1)
    issue_idx(2, 2)

    NB = N_UBATCH

    def loop_body(i, carry):
        for b in range(3):
            bi = 3 * i + b
            sl = b
            cl = (b + 1) % 3

            @pl.when(bi < NB)
            def _():
                wait_idx(bi, sl)
            @pl.when(bi >= 3)
            def _():
                wait_scatter(sl)
            @pl.when(bi < NB)
            def _():
                stash(sl)
                issue_gather(sl)
            @pl.when(bi + 3 < NB)
            def _():
                issue_idx(bi + 3, sl)
            @pl.when(jnp.logical_and(bi >= 2, bi <= NB + 1))
            def _():
                wait_gather(cl)
                compute_batch(cl)
                issue_scatter(cl)
        return carry

    lax.fori_loop(0, (NB + 3) // 3, loop_body, 0)
    plsc.subcore_barrier()

    for j in range(10):
        b = j * 16 + sid
        @pl.when(b < U_BLOCKS)
        def _():
            pltpu.sync_copy(uacc_sh.at[pl.ds(b * 128, 128)],
                            out_hbm.at[cid].at[pl.ds(b * 128, 128)])


def _user_sc(emb, nz_packed, zu):
    mesh = plsc.VectorSubcoreMesh(core_axis_name="c", subcore_axis_name="s",
                                  num_cores=NC, num_subcores=NS)
    ib = lambda: pltpu.VMEM((UB,), jnp.int32)
    fb = lambda: pltpu.VMEM((UB,), jnp.float32)
    db = lambda: pltpu.VMEM((UB, D), jnp.float32)
    scratch = [pltpu.VMEM_SHARED((U_ROWS, D), jnp.float32)]
    scratch += [pltpu.VMEM((3 * UB,), jnp.int32) for _ in range(3)]
    scratch += [ib() for _ in range(3)]
    scratch += [fb() for _ in range(3)]
    scratch += [ib() for _ in range(3)]
    scratch += [db() for _ in range(6)]
    scratch += [pltpu.SemaphoreType.DMA for _ in range(9)]
    f = pl.kernel(
        _user_sc_body,
        out_type=jax.ShapeDtypeStruct((NC, U_ROWS, D), jnp.float32),
        mesh=mesh,
        compiler_params=pltpu.CompilerParams(use_tc_tiling_on_sc=False, needs_layout_passes=False),
        scratch_types=scratch,
    )
    return f(emb, nz_packed, zu)


def _ep_body(ue_ref, w_ref, p_ref, o_ref):
    ue = ue_ref[...]
    w = w_ref[...]
    logits = lax.dot_general(ue, w, (((1,), (1,)), ((), ())),
                             preferred_element_type=jnp.float32)
    col = lax.broadcasted_iota(jnp.int32, logits.shape, 1)
    logits = jnp.where(col < R, logits, -1e30)
    m = jnp.max(logits, axis=-1, keepdims=True)
    ex = jnp.exp(logits - m)
    score = ex / jnp.sum(ex, axis=-1, keepdims=True)
    mult = jnp.dot(score, w, preferred_element_type=jnp.float32)
    p = p_ref[0] + p_ref[1]
    o_ref[...] = p * (1.0 + mult)


def _epilogue(user_emb, w24, partials):
    blk = 1000
    return pl.pallas_call(
        _ep_body,
        grid=(NU // blk,),
        in_specs=[
            pl.BlockSpec((blk, D), lambda i: (i, 0)),
            pl.BlockSpec((24, D), lambda i: (0, 0)),
            pl.BlockSpec((NC, blk, D), lambda i: (0, i, 0)),
        ],
        out_specs=pl.BlockSpec((blk, D), lambda i: (i, 0)),
        out_shape=jax.ShapeDtypeStruct((NU, D), jnp.float32),
    )(user_emb, w24, partials)


def kernel(entity_emb, user_emb, edge_index, edge_type, interact_rows,
           interact_cols, interact_values, weight):
    i32 = jnp.int32
    head = edge_index[0].astype(i32)
    tail = edge_index[1].astype(i32)
    etype = edge_type.astype(i32)

    emb_pad = jnp.pad(entity_emb, ((0, QROWS - NE), (0, 0)))
    wpad = jnp.pad(weight, ((0, 128 - R), (0, 0)))
    q = _compute_q(emb_pad, wpad)
    qflat = q.reshape(-1)

    npad = E_PAD - E_REAL
    head_p = jnp.concatenate([head, jnp.full((npad,), NE, i32)])
    tail_p = jnp.concatenate([tail, jnp.zeros((npad,), i32)])
    y_p = jnp.concatenate([etype, jnp.ones((npad,), i32)])
    edges_packed = jnp.stack(
        [head_p.reshape(NS, N_EBATCH, EB),
         tail_p.reshape(NS, N_EBATCH, EB),
         y_p.reshape(NS, N_EBATCH, EB)], axis=2).reshape(NS, N_EBATCH * 3 * EB)

    emb2 = jnp.stack([entity_emb[:, :HALF], entity_emb[:, HALF:]])
    w2 = jnp.stack([weight[:, :HALF].reshape(-1), weight[:, HALF:].reshape(-1)])

    zr = jnp.zeros((128, HALF), jnp.float32)
    zd = jnp.zeros((128,), jnp.float32)
    num, den = _entity_sc(qflat, emb2, edges_packed, w2, zr, zd)
    entity_agg = _entity_div(num, den)[:NE]

    upad = NNZ_PAD - NNZ_REAL
    cols_p = jnp.concatenate([interact_cols.astype(i32), jnp.zeros((upad,), i32)])
    rows_p = jnp.concatenate([interact_rows.astype(i32), jnp.zeros((upad,), i32)])
    vals_p = jnp.concatenate([interact_values, jnp.zeros((upad,), jnp.float32)])
    nz_packed = jnp.stack(
        [cols_p.reshape(NW, N_UBATCH, UB),
         rows_p.reshape(NW, N_UBATCH, UB),
         lax.bitcast_convert_type(vals_p, i32).reshape(NW, N_UBATCH, UB)],
        axis=2).reshape(NW, N_UBATCH * 3 * UB)
    zu = jnp.zeros((128, D), jnp.float32)
    partials = _user_sc(entity_emb, nz_packed, zu)

    w24 = jnp.pad(weight, ((0, 1), (0, 0)))
    user_agg = _epilogue(user_emb, w24, partials)
    return (entity_agg, user_agg)
